# Initial kernel scaffold; baseline (speedup 1.0000x reference)
import jax
import jax.numpy as jnp
from jax.experimental import pallas as pl


def kernel(x, edge_index, params):
    raise NotImplementedError("write your pallas kernel here")


# speedup vs baseline: 5.2110x; 5.2110x over previous
import functools

import jax
import jax.numpy as jnp
from jax import lax
from jax.experimental import pallas as pl
from jax.experimental.pallas import tpu as pltpu
from jax.experimental.pallas import tpu_sc as plsc

N_NODES = 10000
N_EDGES = 320000
FC = 128
ROW_BLK = 1000

N_TILES = 16
EDGES_PER_TILE = N_EDGES // N_TILES
K_EDGE = 80
N_BLOCKS = EDGES_PER_TILE // K_EDGE
N_PAD = 10240
ROWS_PER_TILE = N_PAD // N_TILES


def _dot(a, b):
    return jax.lax.dot_general(
        a, b, (((1,), (0,)), ((), ())),
        precision=jax.lax.Precision.HIGHEST,
        preferred_element_type=jnp.float32)


def _ln(h, g, b):
    mu = jnp.mean(h, axis=-1, keepdims=True)
    var = jnp.mean((h - mu) ** 2, axis=-1, keepdims=True)
    return (h - mu) / jnp.sqrt(var + 1e-5) * g + b


def _linear_body(x_ref, w_ref, b_ref, o_ref):
    o_ref[...] = _dot(x_ref[...], w_ref[...]) + b_ref[...]


def _linear(x, w, b):
    n, cin = x.shape
    cout = w.shape[1]
    grid = n // ROW_BLK
    return pl.pallas_call(
        _linear_body,
        grid=(grid,),
        in_specs=[
            pl.BlockSpec((ROW_BLK, cin), lambda i: (i, 0)),
            pl.BlockSpec((cin, cout), lambda i: (0, 0)),
            pl.BlockSpec((1, cout), lambda i: (0, 0)),
        ],
        out_specs=pl.BlockSpec((ROW_BLK, cout), lambda i: (i, 0)),
        out_shape=jax.ShapeDtypeStruct((n, cout), jnp.float32),
    )(x, w, b.reshape(1, cout))


def _max_body(x_ref, t_ref, o_ref):
    i = pl.program_id(0)
    logits = (jnp.maximum(x_ref[...], 0.0) + 1e-7) * t_ref[...]
    m = jnp.max(logits).reshape(1, 1)

    @pl.when(i == 0)
    def _():
        o_ref[...] = m

    @pl.when(i > 0)
    def _():
        o_ref[...] = jnp.maximum(o_ref[...], m)


def _global_max(x, t):
    n, c = x.shape
    return pl.pallas_call(
        _max_body,
        grid=(n // ROW_BLK,),
        in_specs=[
            pl.BlockSpec((ROW_BLK, c), lambda i: (i, 0)),
            pl.BlockSpec((1, 1), lambda i: (0, 0)),
        ],
        out_specs=pl.BlockSpec((1, 1), lambda i: (0, 0)),
        out_shape=jax.ShapeDtypeStruct((1, 1), jnp.float32),
    )(x, t)


def _pre_body(x_ref, t_ref, m_ref, *o_refs):
    nc = len(o_refs) // 2
    msg = jnp.maximum(x_ref[...], 0.0) + 1e-7
    logits = msg * t_ref[...]
    e = jnp.exp(logits - m_ref[...])
    p = msg * e
    for k in range(nc):
        o_refs[k][...] = e[:, k * 128:(k + 1) * 128]
        o_refs[nc + k][...] = p[:, k * 128:(k + 1) * 128]


def _pre(x, t, m):
    n, c = x.shape
    nc = c // 128
    outs = pl.pallas_call(
        _pre_body,
        grid=(n // ROW_BLK,),
        in_specs=[
            pl.BlockSpec((ROW_BLK, c), lambda i: (i, 0)),
            pl.BlockSpec((1, 1), lambda i: (0, 0)),
            pl.BlockSpec((1, 1), lambda i: (0, 0)),
        ],
        out_specs=[pl.BlockSpec((ROW_BLK, 128), lambda i: (i, 0))] * (2 * nc),
        out_shape=[jax.ShapeDtypeStruct((n, 128), jnp.float32)] * (2 * nc),
    )(x, t, m)
    return outs


def _post_body(*refs, nc, outer):
    den_refs = refs[:nc]
    num_refs = refs[nc:2 * nc]
    idx = 2 * nc
    x_ref, w1_ref, b1_ref, lng_ref, lnb_ref, w2_ref, b2_ref = refs[idx:idx + 7]
    idx += 7
    if outer:
        og_ref, ob_ref = refs[idx:idx + 2]
        idx += 2
    o_ref = refs[idx]

    den = jnp.concatenate([r[...] for r in den_refs], axis=-1)
    num = jnp.concatenate([r[...] for r in num_refs], axis=-1)
    agg = num / (den + 1e-16)
    out = agg + x_ref[...]
    h = _dot(out, w1_ref[...]) + b1_ref[...]
    h = _ln(h, lng_ref[...], lnb_ref[...])
    h = jnp.maximum(h, 0.0)
    y = _dot(h, w2_ref[...]) + b2_ref[...]
    if outer:
        y = jnp.maximum(_ln(y, og_ref[...], ob_ref[...]), 0.0)
    o_ref[...] = y


def _post(den_chunks, num_chunks, x, p, outer):
    n, c = x.shape
    nc = c // 128
    c2 = 2 * c
    args = list(den_chunks) + list(num_chunks) + [
        x, p['W1'], p['b1'].reshape(1, c2), p['lng'].reshape(1, c2),
        p['lnb'].reshape(1, c2), p['W2'], p['b2'].reshape(1, c)]
    specs = (
        [pl.BlockSpec((ROW_BLK, 128), lambda i: (i, 0))] * (2 * nc) + [
            pl.BlockSpec((ROW_BLK, c), lambda i: (i, 0)),
            pl.BlockSpec((c, c2), lambda i: (0, 0)),
            pl.BlockSpec((1, c2), lambda i: (0, 0)),
            pl.BlockSpec((1, c2), lambda i: (0, 0)),
            pl.BlockSpec((1, c2), lambda i: (0, 0)),
            pl.BlockSpec((c2, c), lambda i: (0, 0)),
            pl.BlockSpec((1, c), lambda i: (0, 0)),
        ])
    if outer is not None:
        og, ob = outer
        args += [og.reshape(1, c), ob.reshape(1, c)]
        specs += [pl.BlockSpec((1, c), lambda i: (0, 0)),
                  pl.BlockSpec((1, c), lambda i: (0, 0))]
    return pl.pallas_call(
        functools.partial(_post_body, nc=nc, outer=outer is not None),
        grid=(n // ROW_BLK,),
        in_specs=specs,
        out_specs=pl.BlockSpec((ROW_BLK, c), lambda i: (i, 0)),
        out_shape=jax.ShapeDtypeStruct((n, c), jnp.float32),
    )(*args)


def _fin_body(x_ref, g_ref, b_ref, w_ref, bl_ref, xx_ref, lg_ref):
    h = jnp.maximum(_ln(x_ref[...], g_ref[...], b_ref[...]), 0.0)
    xx_ref[...] = h
    lg_ref[...] = _dot(h, w_ref[...]) + bl_ref[...]


def _fin(x, g, b, wlin, blin):
    n, c = x.shape
    ncls = wlin.shape[1]
    wpad = jnp.zeros((c, 128), jnp.float32).at[:, :ncls].set(wlin)
    bpad = jnp.zeros((1, 128), jnp.float32).at[0, :ncls].set(blin)
    return pl.pallas_call(
        _fin_body,
        grid=(n // ROW_BLK,),
        in_specs=[
            pl.BlockSpec((ROW_BLK, c), lambda i: (i, 0)),
            pl.BlockSpec((1, c), lambda i: (0, 0)),
            pl.BlockSpec((1, c), lambda i: (0, 0)),
            pl.BlockSpec((c, 128), lambda i: (0, 0)),
            pl.BlockSpec((1, 128), lambda i: (0, 0)),
        ],
        out_specs=[pl.BlockSpec((ROW_BLK, c), lambda i: (i, 0)),
                   pl.BlockSpec((ROW_BLK, 128), lambda i: (i, 0))],
        out_shape=[jax.ShapeDtypeStruct((n, c), jnp.float32),
                   jax.ShapeDtypeStruct((n, 128), jnp.float32)],
    )(x, g.reshape(1, c), b.reshape(1, c), wpad, bpad)


@functools.lru_cache(maxsize=None)
def _make_scatter(nc2):
    half = nc2 // 2
    mesh = plsc.VectorSubcoreMesh(core_axis_name="c", subcore_axis_name="s")

    def body(src_hbm, dst_hbm, zero_hbm, *rest):
        g_refs = rest[:nc2]
        out_refs = rest[nc2:2 * nc2]
        sidx, didx, rows, acc, gsem = rest[2 * nc2:]
        cid = lax.axis_index("c")
        sid = lax.axis_index("s")
        ebase = sid * EDGES_PER_TILE
        rbase = sid * ROWS_PER_TILE

        def do_chunk(g_hbm, out_hbm):
            pltpu.sync_copy(zero_hbm, acc.at[pl.ds(rbase, ROWS_PER_TILE)])
            plsc.subcore_barrier()

            def blk(b, carry):
                base = ebase + b * K_EDGE
                pltpu.sync_copy(src_hbm.at[pl.ds(base, K_EDGE)], sidx.at[0])
                pltpu.sync_copy(dst_hbm.at[pl.ds(base, K_EDGE)], didx.at[0])
                pltpu.async_copy(g_hbm.at[sidx.at[0]], rows, gsem).wait()
                pltpu.sync_copy(rows, acc.at[didx.at[0]], add=True)
                return carry

            lax.fori_loop(0, N_BLOCKS, blk, 0)
            plsc.subcore_barrier()
            pltpu.sync_copy(acc.at[pl.ds(rbase, ROWS_PER_TILE)],
                            out_hbm.at[pl.ds(rbase, ROWS_PER_TILE)])
            plsc.subcore_barrier()

        for side in range(2):
            @pl.when(cid == side)
            def _(side=side):
                for j in range(half):
                    k = side * half + j
                    do_chunk(g_refs[k], out_refs[k])

    return pl.kernel(
        body,
        out_type=[jax.ShapeDtypeStruct((N_PAD, FC), jnp.float32)] * nc2,
        mesh=mesh,
        scratch_types=[
            pltpu.VMEM((1, K_EDGE), jnp.int32),
            pltpu.VMEM((1, K_EDGE), jnp.int32),
            pltpu.VMEM((K_EDGE, FC), jnp.float32),
            pltpu.VMEM_SHARED((N_PAD, FC), jnp.float32),
            pltpu.SemaphoreType.DMA,
        ],
    )


def _genconv(x, src, dst, zeros, p, outer):
    c = x.shape[1]
    nc = c // 128
    t = p['t'].reshape(1, 1)
    m = _global_max(x, t)
    g_list = _pre(x, t, m)
    outs = _make_scatter(2 * nc)(src, dst, zeros, *g_list)
    return _post(outs[:nc], outs[nc:], x, p, outer)


def kernel(x, edge_index, params):
    src = edge_index[0].astype(jnp.int32)
    dst = edge_index[1].astype(jnp.int32)
    zeros = jnp.zeros((ROWS_PER_TILE, FC), jnp.float32)

    cur = _linear(x, params['W0'], params['b0'])
    cur = _genconv(cur, src, dst, zeros, params['conv1'], None)
    for i in range(3):
        p = params['conv%d' % (i + 1)]
        outer = (params['norm%d_g' % (i + 1)], params['norm%d_b' % (i + 1)])
        h = _genconv(cur, src, dst, zeros, p, outer)
        cur = jnp.concatenate([cur, h], axis=-1)
    xx, lg = _fin(cur, params['Ng'], params['Nb'],
                  params['Wlin'], params['blin'])
    return (lg[:, :params['Wlin'].shape[1]], xx)


# speedup vs baseline: 7.0906x; 1.3607x over previous
import functools

import jax
import jax.numpy as jnp
from jax import lax
from jax.experimental import pallas as pl
from jax.experimental.pallas import tpu as pltpu
from jax.experimental.pallas import tpu_sc as plsc

N_NODES = 10000
N_EDGES = 320000
FC = 128
ROW_BLK = 1000

N_TILES = 16
EDGES_PER_TILE = N_EDGES // N_TILES
K_EDGE = 80
N_BLOCKS = EDGES_PER_TILE // K_EDGE
N_PAD = 10240
ROWS_PER_TILE = N_PAD // N_TILES


def _dot(a, b):
    return jax.lax.dot_general(
        a, b, (((1,), (0,)), ((), ())),
        precision=jax.lax.Precision.HIGHEST,
        preferred_element_type=jnp.float32)


def _ln(h, g, b):
    mu = jnp.mean(h, axis=-1, keepdims=True)
    var = jnp.mean((h - mu) ** 2, axis=-1, keepdims=True)
    return (h - mu) / jnp.sqrt(var + 1e-5) * g + b


def _linear_body(x_ref, w_ref, b_ref, o_ref):
    o_ref[...] = _dot(x_ref[...], w_ref[...]) + b_ref[...]


def _linear(x, w, b):
    n, cin = x.shape
    cout = w.shape[1]
    grid = n // ROW_BLK
    return pl.pallas_call(
        _linear_body,
        grid=(grid,),
        in_specs=[
            pl.BlockSpec((ROW_BLK, cin), lambda i: (i, 0)),
            pl.BlockSpec((cin, cout), lambda i: (0, 0)),
            pl.BlockSpec((1, cout), lambda i: (0, 0)),
        ],
        out_specs=pl.BlockSpec((ROW_BLK, cout), lambda i: (i, 0)),
        out_shape=jax.ShapeDtypeStruct((n, cout), jnp.float32),
    )(x, w, b.reshape(1, cout))


def _max_body(x_ref, t_ref, o_ref):
    i = pl.program_id(0)
    logits = (jnp.maximum(x_ref[...], 0.0) + 1e-7) * t_ref[...]
    m = jnp.max(logits).reshape(1, 1)

    @pl.when(i == 0)
    def _():
        o_ref[...] = m

    @pl.when(i > 0)
    def _():
        o_ref[...] = jnp.maximum(o_ref[...], m)


def _global_max(x, t):
    n, c = x.shape
    return pl.pallas_call(
        _max_body,
        grid=(n // ROW_BLK,),
        in_specs=[
            pl.BlockSpec((ROW_BLK, c), lambda i: (i, 0)),
            pl.BlockSpec((1, 1), lambda i: (0, 0)),
        ],
        out_specs=pl.BlockSpec((1, 1), lambda i: (0, 0)),
        out_shape=jax.ShapeDtypeStruct((1, 1), jnp.float32),
    )(x, t)


def _pre_body(x_ref, t_ref, m_ref, *o_refs):
    nc = len(o_refs) // 2
    msg = jnp.maximum(x_ref[...], 0.0) + 1e-7
    logits = msg * t_ref[...]
    e = jnp.exp(logits - m_ref[...])
    p = msg * e
    for k in range(nc):
        o_refs[k][...] = e[:, k * 128:(k + 1) * 128]
        o_refs[nc + k][...] = p[:, k * 128:(k + 1) * 128]


def _pre(x, t, m):
    n, c = x.shape
    nc = c // 128
    outs = pl.pallas_call(
        _pre_body,
        grid=(n // ROW_BLK,),
        in_specs=[
            pl.BlockSpec((ROW_BLK, c), lambda i: (i, 0)),
            pl.BlockSpec((1, 1), lambda i: (0, 0)),
            pl.BlockSpec((1, 1), lambda i: (0, 0)),
        ],
        out_specs=[pl.BlockSpec((ROW_BLK, 128), lambda i: (i, 0))] * (2 * nc),
        out_shape=[jax.ShapeDtypeStruct((n, 128), jnp.float32)] * (2 * nc),
    )(x, t, m)
    return outs


def _post_body(*refs, nc, outer):
    den_refs = refs[:nc]
    num_refs = refs[nc:2 * nc]
    idx = 2 * nc
    x_ref, w1_ref, b1_ref, lng_ref, lnb_ref, w2_ref, b2_ref = refs[idx:idx + 7]
    idx += 7
    if outer:
        og_ref, ob_ref = refs[idx:idx + 2]
        idx += 2
    o_ref = refs[idx]

    den = jnp.concatenate([r[...] for r in den_refs], axis=-1)
    num = jnp.concatenate([r[...] for r in num_refs], axis=-1)
    agg = num / (den + 1e-16)
    out = agg + x_ref[...]
    h = _dot(out, w1_ref[...]) + b1_ref[...]
    h = _ln(h, lng_ref[...], lnb_ref[...])
    h = jnp.maximum(h, 0.0)
    y = _dot(h, w2_ref[...]) + b2_ref[...]
    if outer:
        y = jnp.maximum(_ln(y, og_ref[...], ob_ref[...]), 0.0)
    o_ref[...] = y


def _post(den_chunks, num_chunks, x, p, outer):
    n, c = x.shape
    nc = c // 128
    c2 = 2 * c
    args = list(den_chunks) + list(num_chunks) + [
        x, p['W1'], p['b1'].reshape(1, c2), p['lng'].reshape(1, c2),
        p['lnb'].reshape(1, c2), p['W2'], p['b2'].reshape(1, c)]
    specs = (
        [pl.BlockSpec((ROW_BLK, 128), lambda i: (i, 0))] * (2 * nc) + [
            pl.BlockSpec((ROW_BLK, c), lambda i: (i, 0)),
            pl.BlockSpec((c, c2), lambda i: (0, 0)),
            pl.BlockSpec((1, c2), lambda i: (0, 0)),
            pl.BlockSpec((1, c2), lambda i: (0, 0)),
            pl.BlockSpec((1, c2), lambda i: (0, 0)),
            pl.BlockSpec((c2, c), lambda i: (0, 0)),
            pl.BlockSpec((1, c), lambda i: (0, 0)),
        ])
    if outer is not None:
        og, ob = outer
        args += [og.reshape(1, c), ob.reshape(1, c)]
        specs += [pl.BlockSpec((1, c), lambda i: (0, 0)),
                  pl.BlockSpec((1, c), lambda i: (0, 0))]
    return pl.pallas_call(
        functools.partial(_post_body, nc=nc, outer=outer is not None),
        grid=(n // ROW_BLK,),
        in_specs=specs,
        out_specs=pl.BlockSpec((ROW_BLK, c), lambda i: (i, 0)),
        out_shape=jax.ShapeDtypeStruct((n, c), jnp.float32),
    )(*args)


def _fin_body(x_ref, g_ref, b_ref, w_ref, bl_ref, xx_ref, lg_ref):
    h = jnp.maximum(_ln(x_ref[...], g_ref[...], b_ref[...]), 0.0)
    xx_ref[...] = h
    lg_ref[...] = _dot(h, w_ref[...]) + bl_ref[...]


def _fin(x, g, b, wlin, blin):
    n, c = x.shape
    ncls = wlin.shape[1]
    wpad = jnp.zeros((c, 128), jnp.float32).at[:, :ncls].set(wlin)
    bpad = jnp.zeros((1, 128), jnp.float32).at[0, :ncls].set(blin)
    return pl.pallas_call(
        _fin_body,
        grid=(n // ROW_BLK,),
        in_specs=[
            pl.BlockSpec((ROW_BLK, c), lambda i: (i, 0)),
            pl.BlockSpec((1, c), lambda i: (0, 0)),
            pl.BlockSpec((1, c), lambda i: (0, 0)),
            pl.BlockSpec((c, 128), lambda i: (0, 0)),
            pl.BlockSpec((1, 128), lambda i: (0, 0)),
        ],
        out_specs=[pl.BlockSpec((ROW_BLK, c), lambda i: (i, 0)),
                   pl.BlockSpec((ROW_BLK, 128), lambda i: (i, 0))],
        out_shape=[jax.ShapeDtypeStruct((n, c), jnp.float32),
                   jax.ShapeDtypeStruct((n, 128), jnp.float32)],
    )(x, g.reshape(1, c), b.reshape(1, c), wpad, bpad)


@functools.lru_cache(maxsize=None)
def _make_scatter(nc2):
    half = nc2 // 2
    mesh = plsc.VectorSubcoreMesh(core_axis_name="c", subcore_axis_name="s")

    def body(src_hbm, dst_hbm, zero_hbm, *rest):
        g_refs = rest[:nc2]
        out_refs = rest[nc2:2 * nc2]
        sidx, didx, rows0, rows1, acc, gsem = rest[2 * nc2:]
        cid = lax.axis_index("c")
        sid = lax.axis_index("s")
        rbase = sid * ROWS_PER_TILE

        ebase = sid * EDGES_PER_TILE

        def do_chunk(g_hbm, out_hbm):
            pltpu.sync_copy(zero_hbm, acc.at[pl.ds(rbase, ROWS_PER_TILE)])
            plsc.subcore_barrier()

            def blk(i, carry):
                b0 = ebase + 2 * i * K_EDGE
                pltpu.sync_copy(src_hbm.at[pl.ds(b0, K_EDGE)], sidx.at[0])
                pltpu.sync_copy(dst_hbm.at[pl.ds(b0, K_EDGE)], didx.at[0])
                d0 = pltpu.async_copy(g_hbm.at[sidx.at[0]], rows0, gsem)
                pltpu.sync_copy(src_hbm.at[pl.ds(b0 + K_EDGE, K_EDGE)],
                                sidx.at[1])
                pltpu.sync_copy(dst_hbm.at[pl.ds(b0 + K_EDGE, K_EDGE)],
                                didx.at[1])
                d1 = pltpu.async_copy(g_hbm.at[sidx.at[1]], rows1, gsem)
                d0.wait()
                pltpu.sync_copy(rows0, acc.at[didx.at[0]], add=True)
                d1.wait()
                pltpu.sync_copy(rows1, acc.at[didx.at[1]], add=True)
                return carry

            lax.fori_loop(0, N_BLOCKS // 2, blk, 0)
            plsc.subcore_barrier()
            pltpu.sync_copy(acc.at[pl.ds(rbase, ROWS_PER_TILE)],
                            out_hbm.at[pl.ds(rbase, ROWS_PER_TILE)])
            plsc.subcore_barrier()

        for side in range(2):
            @pl.when(cid == side)
            def _(side=side):
                for j in range(half):
                    k = side * half + j
                    do_chunk(g_refs[k], out_refs[k])

    return pl.kernel(
        body,
        out_type=[jax.ShapeDtypeStruct((N_PAD, FC), jnp.float32)] * nc2,
        mesh=mesh,
        scratch_types=[
            pltpu.VMEM((2, K_EDGE), jnp.int32),
            pltpu.VMEM((2, K_EDGE), jnp.int32),
            pltpu.VMEM((K_EDGE, FC), jnp.float32),
            pltpu.VMEM((K_EDGE, FC), jnp.float32),
            pltpu.VMEM_SHARED((N_PAD, FC), jnp.float32),
            pltpu.SemaphoreType.DMA,
        ],
    )


def _genconv(x, src, dst, zeros, p, outer):
    c = x.shape[1]
    nc = c // 128
    t = p['t'].reshape(1, 1)
    m = _global_max(x, t)
    g_list = _pre(x, t, m)
    outs = _make_scatter(2 * nc)(src, dst, zeros, *g_list)
    return _post(outs[:nc], outs[nc:], x, p, outer)


def kernel(x, edge_index, params):
    src = edge_index[0].astype(jnp.int32)
    dst = edge_index[1].astype(jnp.int32)
    zeros = jnp.zeros((ROWS_PER_TILE, FC), jnp.float32)

    cur = _linear(x, params['W0'], params['b0'])
    cur = _genconv(cur, src, dst, zeros, params['conv1'], None)
    for i in range(3):
        p = params['conv%d' % (i + 1)]
        outer = (params['norm%d_g' % (i + 1)], params['norm%d_b' % (i + 1)])
        h = _genconv(cur, src, dst, zeros, p, outer)
        cur = jnp.concatenate([cur, h], axis=-1)
    xx, lg = _fin(cur, params['Ng'], params['Nb'],
                  params['Wlin'], params['blin'])
    return (lg[:, :params['Wlin'].shape[1]], xx)


# speedup vs baseline: 10.2523x; 1.4459x over previous
import functools

import jax
import jax.numpy as jnp
from jax import lax
from jax.experimental import pallas as pl
from jax.experimental.pallas import tpu as pltpu
from jax.experimental.pallas import tpu_sc as plsc

N_NODES = 10000
N_EDGES = 320000
FC = 128
ROW_BLK = 1000

N_TILES = 16
EDGES_PER_TILE = N_EDGES // N_TILES
K_EDGE = 80
N_BLOCKS = EDGES_PER_TILE // K_EDGE
N_PAD = 10240
ROWS_PER_TILE = N_PAD // N_TILES


def _dot(a, b):
    return jax.lax.dot_general(
        a, b, (((1,), (0,)), ((), ())),
        precision=jax.lax.Precision.HIGHEST,
        preferred_element_type=jnp.float32)


def _ln(h, g, b):
    mu = jnp.mean(h, axis=-1, keepdims=True)
    var = jnp.mean((h - mu) ** 2, axis=-1, keepdims=True)
    return (h - mu) / jnp.sqrt(var + 1e-5) * g + b


def _linear_body(x_ref, w_ref, b_ref, o_ref):
    o_ref[...] = _dot(x_ref[...], w_ref[...]) + b_ref[...]


def _linear(x, w, b):
    n, cin = x.shape
    cout = w.shape[1]
    grid = n // ROW_BLK
    return pl.pallas_call(
        _linear_body,
        grid=(grid,),
        in_specs=[
            pl.BlockSpec((ROW_BLK, cin), lambda i: (i, 0)),
            pl.BlockSpec((cin, cout), lambda i: (0, 0)),
            pl.BlockSpec((1, cout), lambda i: (0, 0)),
        ],
        out_specs=pl.BlockSpec((ROW_BLK, cout), lambda i: (i, 0)),
        out_shape=jax.ShapeDtypeStruct((n, cout), jnp.float32),
    )(x, w, b.reshape(1, cout))


def _max_body(x_ref, t_ref, o_ref):
    i = pl.program_id(0)
    logits = (jnp.maximum(x_ref[...], 0.0) + 1e-7) * t_ref[...]
    m = jnp.max(logits).reshape(1, 1)

    @pl.when(i == 0)
    def _():
        o_ref[...] = m

    @pl.when(i > 0)
    def _():
        o_ref[...] = jnp.maximum(o_ref[...], m)


def _global_max(x, t):
    n, c = x.shape
    return pl.pallas_call(
        _max_body,
        grid=(n // ROW_BLK,),
        in_specs=[
            pl.BlockSpec((ROW_BLK, c), lambda i: (i, 0)),
            pl.BlockSpec((1, 1), lambda i: (0, 0)),
        ],
        out_specs=pl.BlockSpec((1, 1), lambda i: (0, 0)),
        out_shape=jax.ShapeDtypeStruct((1, 1), jnp.float32),
    )(x, t)


def _pre_body(x_ref, t_ref, m_ref, *o_refs):
    nc = len(o_refs) // 2
    msg = jnp.maximum(x_ref[...], 0.0) + 1e-7
    logits = msg * t_ref[...]
    e = jnp.exp(logits - m_ref[...])
    p = msg * e
    for k in range(nc):
        o_refs[k][...] = e[:, k * 128:(k + 1) * 128]
        o_refs[nc + k][...] = p[:, k * 128:(k + 1) * 128]


def _pre(x, t, m):
    n, c = x.shape
    nc = c // 128
    outs = pl.pallas_call(
        _pre_body,
        grid=(n // ROW_BLK,),
        in_specs=[
            pl.BlockSpec((ROW_BLK, c), lambda i: (i, 0)),
            pl.BlockSpec((1, 1), lambda i: (0, 0)),
            pl.BlockSpec((1, 1), lambda i: (0, 0)),
        ],
        out_specs=[pl.BlockSpec((ROW_BLK, 128), lambda i: (i, 0))] * (2 * nc),
        out_shape=[jax.ShapeDtypeStruct((n, 128), jnp.float32)] * (2 * nc),
    )(x, t, m)
    return outs


def _post_body(*refs, nc, outer):
    den_refs = refs[:nc]
    num_refs = refs[nc:2 * nc]
    idx = 2 * nc
    x_ref, w1_ref, b1_ref, lng_ref, lnb_ref, w2_ref, b2_ref = refs[idx:idx + 7]
    idx += 7
    if outer:
        og_ref, ob_ref = refs[idx:idx + 2]
        idx += 2
    o_ref = refs[idx]

    den = jnp.concatenate([r[...] for r in den_refs], axis=-1)
    num = jnp.concatenate([r[...] for r in num_refs], axis=-1)
    agg = num / (den + 1e-16)
    out = agg + x_ref[...]
    h = _dot(out, w1_ref[...]) + b1_ref[...]
    h = _ln(h, lng_ref[...], lnb_ref[...])
    h = jnp.maximum(h, 0.0)
    y = _dot(h, w2_ref[...]) + b2_ref[...]
    if outer:
        y = jnp.maximum(_ln(y, og_ref[...], ob_ref[...]), 0.0)
    o_ref[...] = y


def _post(den_chunks, num_chunks, x, p, outer):
    n, c = x.shape
    nc = c // 128
    c2 = 2 * c
    args = list(den_chunks) + list(num_chunks) + [
        x, p['W1'], p['b1'].reshape(1, c2), p['lng'].reshape(1, c2),
        p['lnb'].reshape(1, c2), p['W2'], p['b2'].reshape(1, c)]
    specs = (
        [pl.BlockSpec((ROW_BLK, 128), lambda i: (i, 0))] * (2 * nc) + [
            pl.BlockSpec((ROW_BLK, c), lambda i: (i, 0)),
            pl.BlockSpec((c, c2), lambda i: (0, 0)),
            pl.BlockSpec((1, c2), lambda i: (0, 0)),
            pl.BlockSpec((1, c2), lambda i: (0, 0)),
            pl.BlockSpec((1, c2), lambda i: (0, 0)),
            pl.BlockSpec((c2, c), lambda i: (0, 0)),
            pl.BlockSpec((1, c), lambda i: (0, 0)),
        ])
    if outer is not None:
        og, ob = outer
        args += [og.reshape(1, c), ob.reshape(1, c)]
        specs += [pl.BlockSpec((1, c), lambda i: (0, 0)),
                  pl.BlockSpec((1, c), lambda i: (0, 0))]
    return pl.pallas_call(
        functools.partial(_post_body, nc=nc, outer=outer is not None),
        grid=(n // ROW_BLK,),
        in_specs=specs,
        out_specs=pl.BlockSpec((ROW_BLK, c), lambda i: (i, 0)),
        out_shape=jax.ShapeDtypeStruct((n, c), jnp.float32),
    )(*args)


def _fin_body(x_ref, g_ref, b_ref, w_ref, bl_ref, xx_ref, lg_ref):
    h = jnp.maximum(_ln(x_ref[...], g_ref[...], b_ref[...]), 0.0)
    xx_ref[...] = h
    lg_ref[...] = _dot(h, w_ref[...]) + bl_ref[...]


def _fin(x, g, b, wlin, blin):
    n, c = x.shape
    ncls = wlin.shape[1]
    wpad = jnp.zeros((c, 128), jnp.float32).at[:, :ncls].set(wlin)
    bpad = jnp.zeros((1, 128), jnp.float32).at[0, :ncls].set(blin)
    return pl.pallas_call(
        _fin_body,
        grid=(n // ROW_BLK,),
        in_specs=[
            pl.BlockSpec((ROW_BLK, c), lambda i: (i, 0)),
            pl.BlockSpec((1, c), lambda i: (0, 0)),
            pl.BlockSpec((1, c), lambda i: (0, 0)),
            pl.BlockSpec((c, 128), lambda i: (0, 0)),
            pl.BlockSpec((1, 128), lambda i: (0, 0)),
        ],
        out_specs=[pl.BlockSpec((ROW_BLK, c), lambda i: (i, 0)),
                   pl.BlockSpec((ROW_BLK, 128), lambda i: (i, 0))],
        out_shape=[jax.ShapeDtypeStruct((n, c), jnp.float32),
                   jax.ShapeDtypeStruct((n, 128), jnp.float32)],
    )(x, g.reshape(1, c), b.reshape(1, c), wpad, bpad)


@functools.lru_cache(maxsize=None)
def _make_scatter(nc2):
    half = nc2 // 2
    mesh = plsc.VectorSubcoreMesh(core_axis_name="c", subcore_axis_name="s")

    def body(src_hbm, dst_hbm, zero_hbm, *rest):
        g_refs = rest[:nc2]
        out_refs = rest[nc2:2 * nc2]
        sidx, didx, rows0, rows1, acc, gsem, isem = rest[2 * nc2:]
        cid = lax.axis_index("c")
        sid = lax.axis_index("s")
        rbase = sid * ROWS_PER_TILE
        ebase = sid * EDGES_PER_TILE
        n_pairs = N_BLOCKS // 2

        def idx_copies(pair, buf):
            base = ebase + pair * 2 * K_EDGE
            return [
                (src_hbm.at[pl.ds(base, K_EDGE)], sidx.at[buf, 0]),
                (src_hbm.at[pl.ds(base + K_EDGE, K_EDGE)], sidx.at[buf, 1]),
                (dst_hbm.at[pl.ds(base, K_EDGE)], didx.at[buf, 0]),
                (dst_hbm.at[pl.ds(base + K_EDGE, K_EDGE)], didx.at[buf, 1]),
            ]

        def idx_prefetch(pair, buf):
            for s, d in idx_copies(pair, buf):
                pltpu.async_copy(s, d, isem)

        def idx_wait(pair, buf):
            for s, d in idx_copies(pair, buf):
                pltpu.make_async_copy(s, d, isem).wait()

        def do_pair(g_hbm, buf, next_pair):
            d0 = pltpu.async_copy(g_hbm.at[sidx.at[buf, 0]], rows0, gsem)
            d1 = pltpu.async_copy(g_hbm.at[sidx.at[buf, 1]], rows1, gsem)
            if next_pair is not None:
                idx_prefetch(next_pair, 1 - buf)
            d0.wait()
            pltpu.sync_copy(rows0, acc.at[didx.at[buf, 0]], add=True)
            d1.wait()
            pltpu.sync_copy(rows1, acc.at[didx.at[buf, 1]], add=True)

        def do_chunk(g_hbm, out_hbm):
            pltpu.sync_copy(zero_hbm, acc.at[pl.ds(rbase, ROWS_PER_TILE)])
            plsc.subcore_barrier()

            idx_prefetch(0, 0)

            def blk(i, carry):
                p0 = 2 * i
                idx_wait(p0, 0)
                do_pair(g_hbm, 0, p0 + 1)
                idx_wait(p0 + 1, 1)
                do_pair(g_hbm, 1, p0 + 2)
                return carry

            lax.fori_loop(0, (n_pairs - 1) // 2, blk, 0)
            idx_wait(n_pairs - 1, 0)
            do_pair(g_hbm, 0, None)
            plsc.subcore_barrier()
            pltpu.sync_copy(acc.at[pl.ds(rbase, ROWS_PER_TILE)],
                            out_hbm.at[pl.ds(rbase, ROWS_PER_TILE)])
            plsc.subcore_barrier()

        for side in range(2):
            @pl.when(cid == side)
            def _(side=side):
                for j in range(half):
                    k = side * half + j
                    do_chunk(g_refs[k], out_refs[k])

    return pl.kernel(
        body,
        out_type=[jax.ShapeDtypeStruct((N_PAD, FC), jnp.float32)] * nc2,
        mesh=mesh,
        scratch_types=[
            pltpu.VMEM((2, 2, K_EDGE), jnp.int32),
            pltpu.VMEM((2, 2, K_EDGE), jnp.int32),
            pltpu.VMEM((K_EDGE, FC), jnp.float32),
            pltpu.VMEM((K_EDGE, FC), jnp.float32),
            pltpu.VMEM_SHARED((N_PAD, FC), jnp.float32),
            pltpu.SemaphoreType.DMA,
            pltpu.SemaphoreType.DMA,
        ],
    )


def _genconv(x, src, dst, zeros, p, outer):
    c = x.shape[1]
    nc = c // 128
    t = p['t'].reshape(1, 1)
    m = _global_max(x, t)
    g_list = _pre(x, t, m)
    outs = _make_scatter(2 * nc)(src, dst, zeros, *g_list)
    return _post(outs[:nc], outs[nc:], x, p, outer)


def kernel(x, edge_index, params):
    src = edge_index[0].astype(jnp.int32)
    dst = edge_index[1].astype(jnp.int32)
    zeros = jnp.zeros((ROWS_PER_TILE, FC), jnp.float32)

    cur = _linear(x, params['W0'], params['b0'])
    cur = _genconv(cur, src, dst, zeros, params['conv1'], None)
    for i in range(3):
        p = params['conv%d' % (i + 1)]
        outer = (params['norm%d_g' % (i + 1)], params['norm%d_b' % (i + 1)])
        h = _genconv(cur, src, dst, zeros, p, outer)
        cur = jnp.concatenate([cur, h], axis=-1)
    xx, lg = _fin(cur, params['Ng'], params['Nb'],
                  params['Wlin'], params['blin'])
    return (lg[:, :params['Wlin'].shape[1]], xx)


# speedup vs baseline: 12.3501x; 1.2046x over previous
import functools

import jax
import jax.numpy as jnp
from jax import lax
from jax.experimental import pallas as pl
from jax.experimental.pallas import tpu as pltpu
from jax.experimental.pallas import tpu_sc as plsc

N_NODES = 10000
N_EDGES = 320000
FC = 128
ROW_BLK = 1000

N_TILES = 16
EDGES_PER_TILE = N_EDGES // N_TILES
K_EDGE = 80
N_BLOCKS = EDGES_PER_TILE // K_EDGE
N_PAD = 10240
ROWS_PER_TILE = N_PAD // N_TILES


def _dot(a, b):
    return jax.lax.dot_general(
        a, b, (((1,), (0,)), ((), ())),
        precision=jax.lax.Precision.HIGHEST,
        preferred_element_type=jnp.float32)


def _ln(h, g, b):
    mu = jnp.mean(h, axis=-1, keepdims=True)
    var = jnp.mean((h - mu) ** 2, axis=-1, keepdims=True)
    return (h - mu) / jnp.sqrt(var + 1e-5) * g + b


def _linear_body(x_ref, w_ref, b_ref, o_ref):
    o_ref[...] = _dot(x_ref[...], w_ref[...]) + b_ref[...]


def _linear(x, w, b):
    n, cin = x.shape
    cout = w.shape[1]
    grid = n // ROW_BLK
    return pl.pallas_call(
        _linear_body,
        grid=(grid,),
        in_specs=[
            pl.BlockSpec((ROW_BLK, cin), lambda i: (i, 0)),
            pl.BlockSpec((cin, cout), lambda i: (0, 0)),
            pl.BlockSpec((1, cout), lambda i: (0, 0)),
        ],
        out_specs=pl.BlockSpec((ROW_BLK, cout), lambda i: (i, 0)),
        out_shape=jax.ShapeDtypeStruct((n, cout), jnp.float32),
    )(x, w, b.reshape(1, cout))


def _max_body(x_ref, t_ref, o_ref):
    i = pl.program_id(0)
    logits = (jnp.maximum(x_ref[...], 0.0) + 1e-7) * t_ref[...]
    m = jnp.max(logits).reshape(1, 1)

    @pl.when(i == 0)
    def _():
        o_ref[...] = m

    @pl.when(i > 0)
    def _():
        o_ref[...] = jnp.maximum(o_ref[...], m)


def _global_max(x, t):
    n, c = x.shape
    return pl.pallas_call(
        _max_body,
        grid=(n // ROW_BLK,),
        in_specs=[
            pl.BlockSpec((ROW_BLK, c), lambda i: (i, 0)),
            pl.BlockSpec((1, 1), lambda i: (0, 0)),
        ],
        out_specs=pl.BlockSpec((1, 1), lambda i: (0, 0)),
        out_shape=jax.ShapeDtypeStruct((1, 1), jnp.float32),
    )(x, t)


def _pre_body(x_ref, t_ref, m_ref, *o_refs):
    nc = len(o_refs) // 2
    msg = jnp.maximum(x_ref[...], 0.0) + 1e-7
    logits = msg * t_ref[...]
    e = jnp.exp(logits - m_ref[...])
    p = msg * e
    for k in range(nc):
        o_refs[k][...] = e[:, k * 128:(k + 1) * 128]
        o_refs[nc + k][...] = p[:, k * 128:(k + 1) * 128]


def _pre(x, t, m):
    n, c = x.shape
    nc = c // 128
    outs = pl.pallas_call(
        _pre_body,
        grid=(n // ROW_BLK,),
        in_specs=[
            pl.BlockSpec((ROW_BLK, c), lambda i: (i, 0)),
            pl.BlockSpec((1, 1), lambda i: (0, 0)),
            pl.BlockSpec((1, 1), lambda i: (0, 0)),
        ],
        out_specs=[pl.BlockSpec((ROW_BLK, 128), lambda i: (i, 0))] * (2 * nc),
        out_shape=[jax.ShapeDtypeStruct((n, 128), jnp.float32)] * (2 * nc),
    )(x, t, m)
    return outs


def _post_body(*refs, nc, outer):
    den_refs = refs[:nc]
    num_refs = refs[nc:2 * nc]
    idx = 2 * nc
    x_ref, w1_ref, b1_ref, lng_ref, lnb_ref, w2_ref, b2_ref = refs[idx:idx + 7]
    idx += 7
    if outer:
        og_ref, ob_ref = refs[idx:idx + 2]
        idx += 2
    o_ref = refs[idx]

    den = jnp.concatenate([r[...] for r in den_refs], axis=-1)
    num = jnp.concatenate([r[...] for r in num_refs], axis=-1)
    agg = num / (den + 1e-16)
    out = agg + x_ref[...]
    h = _dot(out, w1_ref[...]) + b1_ref[...]
    h = _ln(h, lng_ref[...], lnb_ref[...])
    h = jnp.maximum(h, 0.0)
    y = _dot(h, w2_ref[...]) + b2_ref[...]
    if outer:
        y = jnp.maximum(_ln(y, og_ref[...], ob_ref[...]), 0.0)
    o_ref[...] = y


def _post(den_chunks, num_chunks, x, p, outer):
    n, c = x.shape
    nc = c // 128
    c2 = 2 * c
    args = list(den_chunks) + list(num_chunks) + [
        x, p['W1'], p['b1'].reshape(1, c2), p['lng'].reshape(1, c2),
        p['lnb'].reshape(1, c2), p['W2'], p['b2'].reshape(1, c)]
    specs = (
        [pl.BlockSpec((ROW_BLK, 128), lambda i: (i, 0))] * (2 * nc) + [
            pl.BlockSpec((ROW_BLK, c), lambda i: (i, 0)),
            pl.BlockSpec((c, c2), lambda i: (0, 0)),
            pl.BlockSpec((1, c2), lambda i: (0, 0)),
            pl.BlockSpec((1, c2), lambda i: (0, 0)),
            pl.BlockSpec((1, c2), lambda i: (0, 0)),
            pl.BlockSpec((c2, c), lambda i: (0, 0)),
            pl.BlockSpec((1, c), lambda i: (0, 0)),
        ])
    if outer is not None:
        og, ob = outer
        args += [og.reshape(1, c), ob.reshape(1, c)]
        specs += [pl.BlockSpec((1, c), lambda i: (0, 0)),
                  pl.BlockSpec((1, c), lambda i: (0, 0))]
    return pl.pallas_call(
        functools.partial(_post_body, nc=nc, outer=outer is not None),
        grid=(n // ROW_BLK,),
        in_specs=specs,
        out_specs=pl.BlockSpec((ROW_BLK, c), lambda i: (i, 0)),
        out_shape=jax.ShapeDtypeStruct((n, c), jnp.float32),
    )(*args)


def _fin_body(x_ref, g_ref, b_ref, w_ref, bl_ref, xx_ref, lg_ref):
    h = jnp.maximum(_ln(x_ref[...], g_ref[...], b_ref[...]), 0.0)
    xx_ref[...] = h
    lg_ref[...] = _dot(h, w_ref[...]) + bl_ref[...]


def _fin(x, g, b, wlin, blin):
    n, c = x.shape
    ncls = wlin.shape[1]
    wpad = jnp.zeros((c, 128), jnp.float32).at[:, :ncls].set(wlin)
    bpad = jnp.zeros((1, 128), jnp.float32).at[0, :ncls].set(blin)
    return pl.pallas_call(
        _fin_body,
        grid=(n // ROW_BLK,),
        in_specs=[
            pl.BlockSpec((ROW_BLK, c), lambda i: (i, 0)),
            pl.BlockSpec((1, c), lambda i: (0, 0)),
            pl.BlockSpec((1, c), lambda i: (0, 0)),
            pl.BlockSpec((c, 128), lambda i: (0, 0)),
            pl.BlockSpec((1, 128), lambda i: (0, 0)),
        ],
        out_specs=[pl.BlockSpec((ROW_BLK, c), lambda i: (i, 0)),
                   pl.BlockSpec((ROW_BLK, 128), lambda i: (i, 0))],
        out_shape=[jax.ShapeDtypeStruct((n, c), jnp.float32),
                   jax.ShapeDtypeStruct((n, 128), jnp.float32)],
    )(x, g.reshape(1, c), b.reshape(1, c), wpad, bpad)


@functools.lru_cache(maxsize=None)
def _make_scatter(nc2):
    half = nc2 // 2
    mesh = plsc.VectorSubcoreMesh(core_axis_name="c", subcore_axis_name="s")

    def body(src_hbm, dst_hbm, zero_hbm, *rest):
        g_refs = rest[:nc2]
        out_refs = rest[nc2:2 * nc2]
        (sidx, didx, rows0, rows1, rows2, rows3,
         acc, g0, g1, g2, g3, isem) = rest[2 * nc2:]
        cid = lax.axis_index("c")
        sid = lax.axis_index("s")
        rbase = sid * ROWS_PER_TILE
        ebase = sid * EDGES_PER_TILE
        n_pairs = N_BLOCKS // 2

        def idx_copies(pair, buf):
            base = ebase + pair * 2 * K_EDGE
            return [
                (src_hbm.at[pl.ds(base, K_EDGE)], sidx.at[buf, 0]),
                (src_hbm.at[pl.ds(base + K_EDGE, K_EDGE)], sidx.at[buf, 1]),
                (dst_hbm.at[pl.ds(base, K_EDGE)], didx.at[buf, 0]),
                (dst_hbm.at[pl.ds(base + K_EDGE, K_EDGE)], didx.at[buf, 1]),
            ]

        def idx_prefetch(pair, buf):
            for s, d in idx_copies(pair, buf):
                pltpu.async_copy(s, d, isem)

        def idx_wait(pair, buf):
            for s, d in idx_copies(pair, buf):
                pltpu.make_async_copy(s, d, isem).wait()

        def do_chunk(g_hbm, out_hbm):
            def gather(slot, half, buf, sem):
                return pltpu.async_copy(g_hbm.at[sidx.at[slot, half]],
                                        buf, sem)

            def gwait_scatter(slot, half, buf, sem):
                pltpu.make_async_copy(g_hbm.at[sidx.at[slot, half]],
                                      buf, sem).wait()
                pltpu.sync_copy(buf, acc.at[didx.at[slot, half]], add=True)

            pltpu.sync_copy(zero_hbm, acc.at[pl.ds(rbase, ROWS_PER_TILE)])
            plsc.subcore_barrier()

            idx_prefetch(0, 0)
            idx_wait(0, 0)
            gather(0, 0, rows0, g0)
            gather(0, 1, rows1, g1)
            idx_prefetch(1, 1)

            def blk(i, carry):
                p0 = 2 * i
                idx_wait(p0 + 1, 1)
                gather(1, 0, rows2, g2)
                gather(1, 1, rows3, g3)
                gwait_scatter(0, 0, rows0, g0)
                gwait_scatter(0, 1, rows1, g1)

                @pl.when(p0 + 2 < n_pairs)
                def _():
                    idx_prefetch(p0 + 2, 0)
                    idx_wait(p0 + 2, 0)
                    gather(0, 0, rows0, g0)
                    gather(0, 1, rows1, g1)

                gwait_scatter(1, 0, rows2, g2)
                gwait_scatter(1, 1, rows3, g3)

                @pl.when(p0 + 3 < n_pairs)
                def _():
                    idx_prefetch(p0 + 3, 1)
                return carry

            lax.fori_loop(0, n_pairs // 2, blk, 0)
            gwait_scatter(0, 0, rows0, g0)
            gwait_scatter(0, 1, rows1, g1)
            plsc.subcore_barrier()
            pltpu.sync_copy(acc.at[pl.ds(rbase, ROWS_PER_TILE)],
                            out_hbm.at[pl.ds(rbase, ROWS_PER_TILE)])
            plsc.subcore_barrier()

        for side in range(2):
            @pl.when(cid == side)
            def _(side=side):
                for j in range(half):
                    k = side * half + j
                    do_chunk(g_refs[k], out_refs[k])

    return pl.kernel(
        body,
        out_type=[jax.ShapeDtypeStruct((N_PAD, FC), jnp.float32)] * nc2,
        mesh=mesh,
        scratch_types=[
            pltpu.VMEM((2, 2, K_EDGE), jnp.int32),
            pltpu.VMEM((2, 2, K_EDGE), jnp.int32),
            pltpu.VMEM((K_EDGE, FC), jnp.float32),
            pltpu.VMEM((K_EDGE, FC), jnp.float32),
            pltpu.VMEM((K_EDGE, FC), jnp.float32),
            pltpu.VMEM((K_EDGE, FC), jnp.float32),
            pltpu.VMEM_SHARED((N_PAD, FC), jnp.float32),
            pltpu.SemaphoreType.DMA,
            pltpu.SemaphoreType.DMA,
            pltpu.SemaphoreType.DMA,
            pltpu.SemaphoreType.DMA,
            pltpu.SemaphoreType.DMA,
        ],
    )


def _genconv(x, src, dst, zeros, p, outer):
    c = x.shape[1]
    nc = c // 128
    t = p['t'].reshape(1, 1)
    m = _global_max(x, t)
    g_list = _pre(x, t, m)
    outs = _make_scatter(2 * nc)(src, dst, zeros, *g_list)
    return _post(outs[:nc], outs[nc:], x, p, outer)


def kernel(x, edge_index, params):
    src = edge_index[0].astype(jnp.int32)
    dst = edge_index[1].astype(jnp.int32)
    zeros = jnp.zeros((ROWS_PER_TILE, FC), jnp.float32)

    cur = _linear(x, params['W0'], params['b0'])
    cur = _genconv(cur, src, dst, zeros, params['conv1'], None)
    for i in range(3):
        p = params['conv%d' % (i + 1)]
        outer = (params['norm%d_g' % (i + 1)], params['norm%d_b' % (i + 1)])
        h = _genconv(cur, src, dst, zeros, p, outer)
        cur = jnp.concatenate([cur, h], axis=-1)
    xx, lg = _fin(cur, params['Ng'], params['Nb'],
                  params['Wlin'], params['blin'])
    return (lg[:, :params['Wlin'].shape[1]], xx)


# speedup vs baseline: 12.4874x; 1.0111x over previous
import functools

import jax
import jax.numpy as jnp
from jax import lax
from jax.experimental import pallas as pl
from jax.experimental.pallas import tpu as pltpu
from jax.experimental.pallas import tpu_sc as plsc

N_NODES = 10000
N_EDGES = 320000
FC = 128
ROW_BLK = 1000

N_TILES = 16
EDGES_PER_TILE = N_EDGES // N_TILES
K_EDGE = 80
N_BLOCKS = EDGES_PER_TILE // K_EDGE
N_PAD = 10240
ROWS_PER_TILE = N_PAD // N_TILES


def _dot(a, b):
    return jax.lax.dot_general(
        a, b, (((1,), (0,)), ((), ())),
        precision=jax.lax.Precision.HIGHEST,
        preferred_element_type=jnp.float32)


def _ln(h, g, b):
    mu = jnp.mean(h, axis=-1, keepdims=True)
    var = jnp.mean((h - mu) ** 2, axis=-1, keepdims=True)
    return (h - mu) / jnp.sqrt(var + 1e-5) * g + b


def _accum_max(o_ref, y):
    i = pl.program_id(0)
    m = jnp.max(jnp.maximum(y, 0.0)).reshape(1, 1)

    @pl.when(i == 0)
    def _():
        o_ref[...] = m

    @pl.when(i > 0)
    def _():
        o_ref[...] = jnp.maximum(o_ref[...], m)


def _linear_body(x_ref, w_ref, b_ref, o_ref, m_ref):
    y = _dot(x_ref[...], w_ref[...]) + b_ref[...]
    o_ref[...] = y
    _accum_max(m_ref, y)


def _linear(x, w, b):
    n, cin = x.shape
    cout = w.shape[1]
    grid = n // ROW_BLK
    return pl.pallas_call(
        _linear_body,
        grid=(grid,),
        in_specs=[
            pl.BlockSpec((ROW_BLK, cin), lambda i: (i, 0)),
            pl.BlockSpec((cin, cout), lambda i: (0, 0)),
            pl.BlockSpec((1, cout), lambda i: (0, 0)),
        ],
        out_specs=[pl.BlockSpec((ROW_BLK, cout), lambda i: (i, 0)),
                   pl.BlockSpec((1, 1), lambda i: (0, 0))],
        out_shape=[jax.ShapeDtypeStruct((n, cout), jnp.float32),
                   jax.ShapeDtypeStruct((1, 1), jnp.float32)],
    )(x, w, b.reshape(1, cout))


def _pre_body(x_ref, t_ref, m_ref, *o_refs):
    nc = len(o_refs) // 2
    r = jnp.maximum(x_ref[...], 0.0)
    msg = r + 1e-7
    e = jnp.exp((r - m_ref[...]) * t_ref[...])
    p = msg * e
    for k in range(nc):
        o_refs[k][...] = e[:, k * 128:(k + 1) * 128]
        o_refs[nc + k][...] = p[:, k * 128:(k + 1) * 128]


def _pre(x, t, m):
    n, c = x.shape
    nc = c // 128
    outs = pl.pallas_call(
        _pre_body,
        grid=(n // ROW_BLK,),
        in_specs=[
            pl.BlockSpec((ROW_BLK, c), lambda i: (i, 0)),
            pl.BlockSpec((1, 1), lambda i: (0, 0)),
            pl.BlockSpec((1, 1), lambda i: (0, 0)),
        ],
        out_specs=[pl.BlockSpec((ROW_BLK, 128), lambda i: (i, 0))] * (2 * nc),
        out_shape=[jax.ShapeDtypeStruct((n, 128), jnp.float32)] * (2 * nc),
    )(x, t, m)
    return outs


def _post_body(*refs, nc, outer):
    den_refs = refs[:nc]
    num_refs = refs[nc:2 * nc]
    idx = 2 * nc
    x_ref, w1_ref, b1_ref, lng_ref, lnb_ref, w2_ref, b2_ref = refs[idx:idx + 7]
    idx += 7
    if outer:
        og_ref, ob_ref = refs[idx:idx + 2]
        idx += 2
    o_ref, m_ref = refs[idx], refs[idx + 1]

    den = jnp.concatenate([r[...] for r in den_refs], axis=-1)
    num = jnp.concatenate([r[...] for r in num_refs], axis=-1)
    agg = num / (den + 1e-16)
    out = agg + x_ref[...]
    h = _dot(out, w1_ref[...]) + b1_ref[...]
    h = _ln(h, lng_ref[...], lnb_ref[...])
    h = jnp.maximum(h, 0.0)
    y = _dot(h, w2_ref[...]) + b2_ref[...]
    if outer:
        y = jnp.maximum(_ln(y, og_ref[...], ob_ref[...]), 0.0)
    o_ref[...] = y
    _accum_max(m_ref, y)


def _post(den_chunks, num_chunks, x, p, outer):
    n, c = x.shape
    nc = c // 128
    c2 = 2 * c
    args = list(den_chunks) + list(num_chunks) + [
        x, p['W1'], p['b1'].reshape(1, c2), p['lng'].reshape(1, c2),
        p['lnb'].reshape(1, c2), p['W2'], p['b2'].reshape(1, c)]
    specs = (
        [pl.BlockSpec((ROW_BLK, 128), lambda i: (i, 0))] * (2 * nc) + [
            pl.BlockSpec((ROW_BLK, c), lambda i: (i, 0)),
            pl.BlockSpec((c, c2), lambda i: (0, 0)),
            pl.BlockSpec((1, c2), lambda i: (0, 0)),
            pl.BlockSpec((1, c2), lambda i: (0, 0)),
            pl.BlockSpec((1, c2), lambda i: (0, 0)),
            pl.BlockSpec((c2, c), lambda i: (0, 0)),
            pl.BlockSpec((1, c), lambda i: (0, 0)),
        ])
    if outer is not None:
        og, ob = outer
        args += [og.reshape(1, c), ob.reshape(1, c)]
        specs += [pl.BlockSpec((1, c), lambda i: (0, 0)),
                  pl.BlockSpec((1, c), lambda i: (0, 0))]
    return pl.pallas_call(
        functools.partial(_post_body, nc=nc, outer=outer is not None),
        grid=(n // ROW_BLK,),
        in_specs=specs,
        out_specs=[pl.BlockSpec((ROW_BLK, c), lambda i: (i, 0)),
                   pl.BlockSpec((1, 1), lambda i: (0, 0))],
        out_shape=[jax.ShapeDtypeStruct((n, c), jnp.float32),
                   jax.ShapeDtypeStruct((1, 1), jnp.float32)],
    )(*args)


def _fin_body(x_ref, g_ref, b_ref, w_ref, bl_ref, xx_ref, lg_ref):
    h = jnp.maximum(_ln(x_ref[...], g_ref[...], b_ref[...]), 0.0)
    xx_ref[...] = h
    lg_ref[...] = _dot(h, w_ref[...]) + bl_ref[...]


def _fin(x, g, b, wlin, blin):
    n, c = x.shape
    ncls = wlin.shape[1]
    wpad = jnp.zeros((c, 128), jnp.float32).at[:, :ncls].set(wlin)
    bpad = jnp.zeros((1, 128), jnp.float32).at[0, :ncls].set(blin)
    return pl.pallas_call(
        _fin_body,
        grid=(n // ROW_BLK,),
        in_specs=[
            pl.BlockSpec((ROW_BLK, c), lambda i: (i, 0)),
            pl.BlockSpec((1, c), lambda i: (0, 0)),
            pl.BlockSpec((1, c), lambda i: (0, 0)),
            pl.BlockSpec((c, 128), lambda i: (0, 0)),
            pl.BlockSpec((1, 128), lambda i: (0, 0)),
        ],
        out_specs=[pl.BlockSpec((ROW_BLK, c), lambda i: (i, 0)),
                   pl.BlockSpec((ROW_BLK, 128), lambda i: (i, 0))],
        out_shape=[jax.ShapeDtypeStruct((n, c), jnp.float32),
                   jax.ShapeDtypeStruct((n, 128), jnp.float32)],
    )(x, g.reshape(1, c), b.reshape(1, c), wpad, bpad)


@functools.lru_cache(maxsize=None)
def _make_scatter(nc2):
    half = nc2 // 2
    mesh = plsc.VectorSubcoreMesh(core_axis_name="c", subcore_axis_name="s")

    def body(src_hbm, dst_hbm, zero_hbm, *rest):
        g_refs = rest[:nc2]
        out_refs = rest[nc2:2 * nc2]
        (sidx, didx, rows0, rows1, rows2, rows3,
         acc, g0, g1, g2, g3, isem) = rest[2 * nc2:]
        cid = lax.axis_index("c")
        sid = lax.axis_index("s")
        rbase = sid * ROWS_PER_TILE
        ebase = sid * EDGES_PER_TILE
        n_pairs = N_BLOCKS // 2

        def idx_copies(pair, buf):
            base = ebase + pair * 2 * K_EDGE
            return [
                (src_hbm.at[pl.ds(base, K_EDGE)], sidx.at[buf, 0]),
                (src_hbm.at[pl.ds(base + K_EDGE, K_EDGE)], sidx.at[buf, 1]),
                (dst_hbm.at[pl.ds(base, K_EDGE)], didx.at[buf, 0]),
                (dst_hbm.at[pl.ds(base + K_EDGE, K_EDGE)], didx.at[buf, 1]),
            ]

        def idx_prefetch(pair, buf):
            for s, d in idx_copies(pair, buf):
                pltpu.async_copy(s, d, isem)

        def idx_wait(pair, buf):
            for s, d in idx_copies(pair, buf):
                pltpu.make_async_copy(s, d, isem).wait()

        def do_chunk(g_hbm, out_hbm):
            def gather(slot, half, buf, sem):
                return pltpu.async_copy(g_hbm.at[sidx.at[slot, half]],
                                        buf, sem)

            def gwait_scatter(slot, half, buf, sem):
                pltpu.make_async_copy(g_hbm.at[sidx.at[slot, half]],
                                      buf, sem).wait()
                pltpu.sync_copy(buf, acc.at[didx.at[slot, half]], add=True)

            pltpu.sync_copy(zero_hbm, acc.at[pl.ds(rbase, ROWS_PER_TILE)])
            plsc.subcore_barrier()

            idx_prefetch(0, 0)
            idx_wait(0, 0)
            gather(0, 0, rows0, g0)
            gather(0, 1, rows1, g1)
            idx_prefetch(1, 1)

            def blk(i, carry):
                p0 = 2 * i
                idx_wait(p0 + 1, 1)
                gather(1, 0, rows2, g2)
                gather(1, 1, rows3, g3)
                gwait_scatter(0, 0, rows0, g0)
                gwait_scatter(0, 1, rows1, g1)

                @pl.when(p0 + 2 < n_pairs)
                def _():
                    idx_prefetch(p0 + 2, 0)
                    idx_wait(p0 + 2, 0)
                    gather(0, 0, rows0, g0)
                    gather(0, 1, rows1, g1)

                gwait_scatter(1, 0, rows2, g2)
                gwait_scatter(1, 1, rows3, g3)

                @pl.when(p0 + 3 < n_pairs)
                def _():
                    idx_prefetch(p0 + 3, 1)
                return carry

            lax.fori_loop(0, n_pairs // 2, blk, 0)
            gwait_scatter(0, 0, rows0, g0)
            gwait_scatter(0, 1, rows1, g1)
            plsc.subcore_barrier()
            pltpu.sync_copy(acc.at[pl.ds(rbase, ROWS_PER_TILE)],
                            out_hbm.at[pl.ds(rbase, ROWS_PER_TILE)])
            plsc.subcore_barrier()

        for side in range(2):
            @pl.when(cid == side)
            def _(side=side):
                for j in range(half):
                    k = side * half + j
                    do_chunk(g_refs[k], out_refs[k])

    return pl.kernel(
        body,
        out_type=[jax.ShapeDtypeStruct((N_PAD, FC), jnp.float32)] * nc2,
        mesh=mesh,
        scratch_types=[
            pltpu.VMEM((2, 2, K_EDGE), jnp.int32),
            pltpu.VMEM((2, 2, K_EDGE), jnp.int32),
            pltpu.VMEM((K_EDGE, FC), jnp.float32),
            pltpu.VMEM((K_EDGE, FC), jnp.float32),
            pltpu.VMEM((K_EDGE, FC), jnp.float32),
            pltpu.VMEM((K_EDGE, FC), jnp.float32),
            pltpu.VMEM_SHARED((N_PAD, FC), jnp.float32),
            pltpu.SemaphoreType.DMA,
            pltpu.SemaphoreType.DMA,
            pltpu.SemaphoreType.DMA,
            pltpu.SemaphoreType.DMA,
            pltpu.SemaphoreType.DMA,
        ],
    )


def _genconv(x, m, src, dst, zeros, p, outer):
    c = x.shape[1]
    nc = c // 128
    t = p['t'].reshape(1, 1)
    g_list = _pre(x, t, m)
    outs = _make_scatter(2 * nc)(src, dst, zeros, *g_list)
    return _post(outs[:nc], outs[nc:], x, p, outer)


def kernel(x, edge_index, params):
    src = edge_index[0].astype(jnp.int32)
    dst = edge_index[1].astype(jnp.int32)
    zeros = jnp.zeros((ROWS_PER_TILE, FC), jnp.float32)

    cur, m = _linear(x, params['W0'], params['b0'])
    cur, m = _genconv(cur, m, src, dst, zeros, params['conv1'], None)
    for i in range(3):
        p = params['conv%d' % (i + 1)]
        outer = (params['norm%d_g' % (i + 1)], params['norm%d_b' % (i + 1)])
        h, mh = _genconv(cur, m, src, dst, zeros, p, outer)
        cur = jnp.concatenate([cur, h], axis=-1)
        m = jnp.maximum(m, mh)
    xx, lg = _fin(cur, params['Ng'], params['Nb'],
                  params['Wlin'], params['blin'])
    return (lg[:, :params['Wlin'].shape[1]], xx)


# speedup vs baseline: 13.9644x; 1.1183x over previous
import functools

import jax
import jax.numpy as jnp
from jax import lax
from jax.experimental import pallas as pl
from jax.experimental.pallas import tpu as pltpu
from jax.experimental.pallas import tpu_sc as plsc

N_NODES = 10000
N_EDGES = 320000
FC = 128
ROW_BLK = 1000

N_TILES = 16
EDGES_PER_TILE = N_EDGES // N_TILES
K_EDGE = 80
N_BLOCKS = EDGES_PER_TILE // K_EDGE
N_PAD = 10240
ROWS_PER_TILE = N_PAD // N_TILES


def _dot(a, b):
    return jax.lax.dot_general(
        a, b, (((1,), (0,)), ((), ())),
        precision=jax.lax.Precision.DEFAULT,
        preferred_element_type=jnp.float32)


def _ln(h, g, b):
    mu = jnp.mean(h, axis=-1, keepdims=True)
    var = jnp.mean((h - mu) ** 2, axis=-1, keepdims=True)
    return (h - mu) / jnp.sqrt(var + 1e-5) * g + b


def _accum_max(o_ref, y):
    i = pl.program_id(0)
    m = jnp.max(jnp.maximum(y, 0.0)).reshape(1, 1)

    @pl.when(i == 0)
    def _():
        o_ref[...] = m

    @pl.when(i > 0)
    def _():
        o_ref[...] = jnp.maximum(o_ref[...], m)


def _linear_body(x_ref, w_ref, b_ref, o_ref, m_ref):
    y = _dot(x_ref[...], w_ref[...]) + b_ref[...]
    o_ref[...] = y
    _accum_max(m_ref, y)


def _linear(x, w, b):
    n, cin = x.shape
    cout = w.shape[1]
    grid = n // ROW_BLK
    return pl.pallas_call(
        _linear_body,
        grid=(grid,),
        in_specs=[
            pl.BlockSpec((ROW_BLK, cin), lambda i: (i, 0)),
            pl.BlockSpec((cin, cout), lambda i: (0, 0)),
            pl.BlockSpec((1, cout), lambda i: (0, 0)),
        ],
        out_specs=[pl.BlockSpec((ROW_BLK, cout), lambda i: (i, 0)),
                   pl.BlockSpec((1, 1), lambda i: (0, 0))],
        out_shape=[jax.ShapeDtypeStruct((n, cout), jnp.float32),
                   jax.ShapeDtypeStruct((1, 1), jnp.float32)],
    )(x, w, b.reshape(1, cout))


def _pre_body(x_ref, t_ref, m_ref, *o_refs):
    nc = len(o_refs) // 2
    r = jnp.maximum(x_ref[...], 0.0)
    msg = r + 1e-7
    e = jnp.exp((r - m_ref[...]) * t_ref[...])
    p = msg * e
    for k in range(nc):
        o_refs[k][...] = e[:, k * 128:(k + 1) * 128]
        o_refs[nc + k][...] = p[:, k * 128:(k + 1) * 128]


def _pre(x, t, m):
    n, c = x.shape
    nc = c // 128
    outs = pl.pallas_call(
        _pre_body,
        grid=(n // ROW_BLK,),
        in_specs=[
            pl.BlockSpec((ROW_BLK, c), lambda i: (i, 0)),
            pl.BlockSpec((1, 1), lambda i: (0, 0)),
            pl.BlockSpec((1, 1), lambda i: (0, 0)),
        ],
        out_specs=[pl.BlockSpec((ROW_BLK, 128), lambda i: (i, 0))] * (2 * nc),
        out_shape=[jax.ShapeDtypeStruct((n, 128), jnp.float32)] * (2 * nc),
    )(x, t, m)
    return outs


def _post_body(*refs, nc, outer):
    den_refs = refs[:nc]
    num_refs = refs[nc:2 * nc]
    idx = 2 * nc
    x_ref, w1_ref, b1_ref, lng_ref, lnb_ref, w2_ref, b2_ref = refs[idx:idx + 7]
    idx += 7
    if outer:
        og_ref, ob_ref = refs[idx:idx + 2]
        idx += 2
    o_ref, m_ref = refs[idx], refs[idx + 1]

    den = jnp.concatenate([r[...] for r in den_refs], axis=-1)
    num = jnp.concatenate([r[...] for r in num_refs], axis=-1)
    agg = num / (den + 1e-16)
    out = agg + x_ref[...]
    h = _dot(out, w1_ref[...]) + b1_ref[...]
    h = _ln(h, lng_ref[...], lnb_ref[...])
    h = jnp.maximum(h, 0.0)
    y = _dot(h, w2_ref[...]) + b2_ref[...]
    if outer:
        y = jnp.maximum(_ln(y, og_ref[...], ob_ref[...]), 0.0)
    o_ref[...] = y
    _accum_max(m_ref, y)


def _post(den_chunks, num_chunks, x, p, outer):
    n, c = x.shape
    nc = c // 128
    c2 = 2 * c
    args = list(den_chunks) + list(num_chunks) + [
        x, p['W1'], p['b1'].reshape(1, c2), p['lng'].reshape(1, c2),
        p['lnb'].reshape(1, c2), p['W2'], p['b2'].reshape(1, c)]
    specs = (
        [pl.BlockSpec((ROW_BLK, 128), lambda i: (i, 0))] * (2 * nc) + [
            pl.BlockSpec((ROW_BLK, c), lambda i: (i, 0)),
            pl.BlockSpec((c, c2), lambda i: (0, 0)),
            pl.BlockSpec((1, c2), lambda i: (0, 0)),
            pl.BlockSpec((1, c2), lambda i: (0, 0)),
            pl.BlockSpec((1, c2), lambda i: (0, 0)),
            pl.BlockSpec((c2, c), lambda i: (0, 0)),
            pl.BlockSpec((1, c), lambda i: (0, 0)),
        ])
    if outer is not None:
        og, ob = outer
        args += [og.reshape(1, c), ob.reshape(1, c)]
        specs += [pl.BlockSpec((1, c), lambda i: (0, 0)),
                  pl.BlockSpec((1, c), lambda i: (0, 0))]
    return pl.pallas_call(
        functools.partial(_post_body, nc=nc, outer=outer is not None),
        grid=(n // ROW_BLK,),
        in_specs=specs,
        out_specs=[pl.BlockSpec((ROW_BLK, c), lambda i: (i, 0)),
                   pl.BlockSpec((1, 1), lambda i: (0, 0))],
        out_shape=[jax.ShapeDtypeStruct((n, c), jnp.float32),
                   jax.ShapeDtypeStruct((1, 1), jnp.float32)],
    )(*args)


def _fin_body(x_ref, g_ref, b_ref, w_ref, bl_ref, xx_ref, lg_ref):
    h = jnp.maximum(_ln(x_ref[...], g_ref[...], b_ref[...]), 0.0)
    xx_ref[...] = h
    lg_ref[...] = _dot(h, w_ref[...]) + bl_ref[...]


def _fin(x, g, b, wlin, blin):
    n, c = x.shape
    ncls = wlin.shape[1]
    wpad = jnp.zeros((c, 128), jnp.float32).at[:, :ncls].set(wlin)
    bpad = jnp.zeros((1, 128), jnp.float32).at[0, :ncls].set(blin)
    return pl.pallas_call(
        _fin_body,
        grid=(n // ROW_BLK,),
        in_specs=[
            pl.BlockSpec((ROW_BLK, c), lambda i: (i, 0)),
            pl.BlockSpec((1, c), lambda i: (0, 0)),
            pl.BlockSpec((1, c), lambda i: (0, 0)),
            pl.BlockSpec((c, 128), lambda i: (0, 0)),
            pl.BlockSpec((1, 128), lambda i: (0, 0)),
        ],
        out_specs=[pl.BlockSpec((ROW_BLK, c), lambda i: (i, 0)),
                   pl.BlockSpec((ROW_BLK, 128), lambda i: (i, 0))],
        out_shape=[jax.ShapeDtypeStruct((n, c), jnp.float32),
                   jax.ShapeDtypeStruct((n, 128), jnp.float32)],
    )(x, g.reshape(1, c), b.reshape(1, c), wpad, bpad)


@functools.lru_cache(maxsize=None)
def _make_scatter(nc2):
    half = nc2 // 2
    mesh = plsc.VectorSubcoreMesh(core_axis_name="c", subcore_axis_name="s")

    def body(src_hbm, dst_hbm, zero_hbm, *rest):
        g_refs = rest[:nc2]
        out_refs = rest[nc2:2 * nc2]
        (sidx, didx, rows0, rows1, rows2, rows3,
         acc, g0, g1, g2, g3, isem) = rest[2 * nc2:]
        cid = lax.axis_index("c")
        sid = lax.axis_index("s")
        rbase = sid * ROWS_PER_TILE
        ebase = sid * EDGES_PER_TILE
        n_pairs = N_BLOCKS // 2

        def idx_copies(pair, buf):
            base = ebase + pair * 2 * K_EDGE
            return [
                (src_hbm.at[pl.ds(base, K_EDGE)], sidx.at[buf, 0]),
                (src_hbm.at[pl.ds(base + K_EDGE, K_EDGE)], sidx.at[buf, 1]),
                (dst_hbm.at[pl.ds(base, K_EDGE)], didx.at[buf, 0]),
                (dst_hbm.at[pl.ds(base + K_EDGE, K_EDGE)], didx.at[buf, 1]),
            ]

        def idx_prefetch(pair, buf):
            for s, d in idx_copies(pair, buf):
                pltpu.async_copy(s, d, isem)

        def idx_wait(pair, buf):
            for s, d in idx_copies(pair, buf):
                pltpu.make_async_copy(s, d, isem).wait()

        def do_chunk(g_hbm, out_hbm):
            def gather(slot, half, buf, sem):
                return pltpu.async_copy(g_hbm.at[sidx.at[slot, half]],
                                        buf, sem)

            def gwait_scatter(slot, half, buf, sem):
                pltpu.make_async_copy(g_hbm.at[sidx.at[slot, half]],
                                      buf, sem).wait()
                pltpu.sync_copy(buf, acc.at[didx.at[slot, half]], add=True)

            pltpu.sync_copy(zero_hbm, acc.at[pl.ds(rbase, ROWS_PER_TILE)])
            plsc.subcore_barrier()

            idx_prefetch(0, 0)
            idx_wait(0, 0)
            gather(0, 0, rows0, g0)
            gather(0, 1, rows1, g1)
            idx_prefetch(1, 1)

            def blk(i, carry):
                p0 = 2 * i
                idx_wait(p0 + 1, 1)
                gather(1, 0, rows2, g2)
                gather(1, 1, rows3, g3)
                gwait_scatter(0, 0, rows0, g0)
                gwait_scatter(0, 1, rows1, g1)

                @pl.when(p0 + 2 < n_pairs)
                def _():
                    idx_prefetch(p0 + 2, 0)
                    idx_wait(p0 + 2, 0)
                    gather(0, 0, rows0, g0)
                    gather(0, 1, rows1, g1)

                gwait_scatter(1, 0, rows2, g2)
                gwait_scatter(1, 1, rows3, g3)

                @pl.when(p0 + 3 < n_pairs)
                def _():
                    idx_prefetch(p0 + 3, 1)
                return carry

            lax.fori_loop(0, n_pairs // 2, blk, 0)
            gwait_scatter(0, 0, rows0, g0)
            gwait_scatter(0, 1, rows1, g1)
            plsc.subcore_barrier()
            pltpu.sync_copy(acc.at[pl.ds(rbase, ROWS_PER_TILE)],
                            out_hbm.at[pl.ds(rbase, ROWS_PER_TILE)])
            plsc.subcore_barrier()

        for side in range(2):
            @pl.when(cid == side)
            def _(side=side):
                for j in range(half):
                    k = side * half + j
                    do_chunk(g_refs[k], out_refs[k])

    return pl.kernel(
        body,
        out_type=[jax.ShapeDtypeStruct((N_PAD, FC), jnp.float32)] * nc2,
        mesh=mesh,
        scratch_types=[
            pltpu.VMEM((2, 2, K_EDGE), jnp.int32),
            pltpu.VMEM((2, 2, K_EDGE), jnp.int32),
            pltpu.VMEM((K_EDGE, FC), jnp.float32),
            pltpu.VMEM((K_EDGE, FC), jnp.float32),
            pltpu.VMEM((K_EDGE, FC), jnp.float32),
            pltpu.VMEM((K_EDGE, FC), jnp.float32),
            pltpu.VMEM_SHARED((N_PAD, FC), jnp.float32),
            pltpu.SemaphoreType.DMA,
            pltpu.SemaphoreType.DMA,
            pltpu.SemaphoreType.DMA,
            pltpu.SemaphoreType.DMA,
            pltpu.SemaphoreType.DMA,
        ],
    )


def _genconv(x, m, src, dst, zeros, p, outer):
    c = x.shape[1]
    nc = c // 128
    t = p['t'].reshape(1, 1)
    g_list = _pre(x, t, m)
    outs = _make_scatter(2 * nc)(src, dst, zeros, *g_list)
    return _post(outs[:nc], outs[nc:], x, p, outer)


def kernel(x, edge_index, params):
    src = edge_index[0].astype(jnp.int32)
    dst = edge_index[1].astype(jnp.int32)
    zeros = jnp.zeros((ROWS_PER_TILE, FC), jnp.float32)

    cur, m = _linear(x, params['W0'], params['b0'])
    cur, m = _genconv(cur, m, src, dst, zeros, params['conv1'], None)
    for i in range(3):
        p = params['conv%d' % (i + 1)]
        outer = (params['norm%d_g' % (i + 1)], params['norm%d_b' % (i + 1)])
        h, mh = _genconv(cur, m, src, dst, zeros, p, outer)
        cur = jnp.concatenate([cur, h], axis=-1)
        m = jnp.maximum(m, mh)
    xx, lg = _fin(cur, params['Ng'], params['Nb'],
                  params['Wlin'], params['blin'])
    return (lg[:, :params['Wlin'].shape[1]], xx)


# speedup vs baseline: 14.2458x; 1.0202x over previous
import functools

import jax
import jax.numpy as jnp
from jax import lax
from jax.experimental import pallas as pl
from jax.experimental.pallas import tpu as pltpu
from jax.experimental.pallas import tpu_sc as plsc

N_NODES = 10000
N_EDGES = 320000
FC = 128
ROW_BLK = 1000

N_TILES = 16
EDGES_PER_TILE = N_EDGES // N_TILES
K_EDGE = 80
N_BLOCKS = EDGES_PER_TILE // K_EDGE
N_PAD = 10240
ROWS_PER_TILE = N_PAD // N_TILES


def _dot(a, b):
    return jax.lax.dot_general(
        a, b, (((1,), (0,)), ((), ())),
        precision=jax.lax.Precision.DEFAULT,
        preferred_element_type=jnp.float32)


def _ln(h, g, b):
    mu = jnp.mean(h, axis=-1, keepdims=True)
    var = jnp.mean((h - mu) ** 2, axis=-1, keepdims=True)
    return (h - mu) / jnp.sqrt(var + 1e-5) * g + b


def _accum_max(o_ref, y):
    i = pl.program_id(0)
    m = jnp.max(jnp.maximum(y, 0.0)).reshape(1, 1)

    @pl.when(i == 0)
    def _():
        o_ref[...] = m

    @pl.when(i > 0)
    def _():
        o_ref[...] = jnp.maximum(o_ref[...], m)


def _linear_body(x_ref, w_ref, b_ref, o_ref, m_ref):
    y = _dot(x_ref[...], w_ref[...]) + b_ref[...]
    o_ref[...] = y
    _accum_max(m_ref, y)


def _linear(x, w, b):
    n, cin = x.shape
    cout = w.shape[1]
    grid = n // ROW_BLK
    return pl.pallas_call(
        _linear_body,
        grid=(grid,),
        in_specs=[
            pl.BlockSpec((ROW_BLK, cin), lambda i: (i, 0)),
            pl.BlockSpec((cin, cout), lambda i: (0, 0)),
            pl.BlockSpec((1, cout), lambda i: (0, 0)),
        ],
        out_specs=[pl.BlockSpec((ROW_BLK, cout), lambda i: (i, 0)),
                   pl.BlockSpec((1, 1), lambda i: (0, 0))],
        out_shape=[jax.ShapeDtypeStruct((n, cout), jnp.float32),
                   jax.ShapeDtypeStruct((1, 1), jnp.float32)],
    )(x, w, b.reshape(1, cout))


def _pre_body(x_ref, t_ref, m_ref, *o_refs):
    nc = len(o_refs) // 2
    r = jnp.maximum(x_ref[...], 0.0)
    msg = r + 1e-7
    e = jnp.exp((r - m_ref[...]) * t_ref[...])
    p = msg * e
    for k in range(nc):
        o_refs[k][...] = e[:, k * 128:(k + 1) * 128]
        o_refs[nc + k][...] = p[:, k * 128:(k + 1) * 128]


def _pre(x, t, m):
    n, c = x.shape
    nc = c // 128
    outs = pl.pallas_call(
        _pre_body,
        grid=(n // ROW_BLK,),
        in_specs=[
            pl.BlockSpec((ROW_BLK, c), lambda i: (i, 0)),
            pl.BlockSpec((1, 1), lambda i: (0, 0)),
            pl.BlockSpec((1, 1), lambda i: (0, 0)),
        ],
        out_specs=[pl.BlockSpec((ROW_BLK, 128), lambda i: (i, 0))] * (2 * nc),
        out_shape=[jax.ShapeDtypeStruct((n, 128), jnp.float32)] * (2 * nc),
    )(x, t, m)
    return outs


def _post_body(*refs, nc, outer, final):
    den_refs = refs[:nc]
    num_refs = refs[nc:2 * nc]
    idx = 2 * nc
    x_ref, w1_ref, b1_ref, lng_ref, lnb_ref, w2_ref, b2_ref = refs[idx:idx + 7]
    idx += 7
    if outer:
        og_ref, ob_ref = refs[idx:idx + 2]
        idx += 2
    if final:
        ng_ref, nb_ref, wl_ref, bl_ref = refs[idx:idx + 4]
        idx += 4
        xx_ref, lg_ref = refs[idx], refs[idx + 1]
    else:
        o_ref, m_ref = refs[idx], refs[idx + 1]

    den = jnp.concatenate([r[...] for r in den_refs], axis=-1)
    num = jnp.concatenate([r[...] for r in num_refs], axis=-1)
    agg = num / (den + 1e-16)
    out = agg + x_ref[...]
    h = _dot(out, w1_ref[...]) + b1_ref[...]
    h = _ln(h, lng_ref[...], lnb_ref[...])
    h = jnp.maximum(h, 0.0)
    y = _dot(h, w2_ref[...]) + b2_ref[...]
    if outer:
        y = jnp.maximum(_ln(y, og_ref[...], ob_ref[...]), 0.0)
    if final:
        cat = jnp.concatenate([x_ref[...], y], axis=-1)
        z = jnp.maximum(_ln(cat, ng_ref[...], nb_ref[...]), 0.0)
        xx_ref[...] = z
        lg_ref[...] = _dot(z, wl_ref[...]) + bl_ref[...]
    else:
        o_ref[...] = y
        _accum_max(m_ref, y)


def _post(den_chunks, num_chunks, x, p, outer, final=None):
    n, c = x.shape
    nc = c // 128
    c2 = 2 * c
    args = list(den_chunks) + list(num_chunks) + [
        x, p['W1'], p['b1'].reshape(1, c2), p['lng'].reshape(1, c2),
        p['lnb'].reshape(1, c2), p['W2'], p['b2'].reshape(1, c)]
    specs = (
        [pl.BlockSpec((ROW_BLK, 128), lambda i: (i, 0))] * (2 * nc) + [
            pl.BlockSpec((ROW_BLK, c), lambda i: (i, 0)),
            pl.BlockSpec((c, c2), lambda i: (0, 0)),
            pl.BlockSpec((1, c2), lambda i: (0, 0)),
            pl.BlockSpec((1, c2), lambda i: (0, 0)),
            pl.BlockSpec((1, c2), lambda i: (0, 0)),
            pl.BlockSpec((c2, c), lambda i: (0, 0)),
            pl.BlockSpec((1, c), lambda i: (0, 0)),
        ])
    if outer is not None:
        og, ob = outer
        args += [og.reshape(1, c), ob.reshape(1, c)]
        specs += [pl.BlockSpec((1, c), lambda i: (0, 0)),
                  pl.BlockSpec((1, c), lambda i: (0, 0))]
    if final is not None:
        ng, nb, wpad, bpad = final
        cc = 2 * c
        args += [ng.reshape(1, cc), nb.reshape(1, cc), wpad, bpad]
        specs += [pl.BlockSpec((1, cc), lambda i: (0, 0)),
                  pl.BlockSpec((1, cc), lambda i: (0, 0)),
                  pl.BlockSpec((cc, 128), lambda i: (0, 0)),
                  pl.BlockSpec((1, 128), lambda i: (0, 0))]
        out_specs = [pl.BlockSpec((ROW_BLK, cc), lambda i: (i, 0)),
                     pl.BlockSpec((ROW_BLK, 128), lambda i: (i, 0))]
        out_shape = [jax.ShapeDtypeStruct((n, cc), jnp.float32),
                     jax.ShapeDtypeStruct((n, 128), jnp.float32)]
    else:
        out_specs = [pl.BlockSpec((ROW_BLK, c), lambda i: (i, 0)),
                     pl.BlockSpec((1, 1), lambda i: (0, 0))]
        out_shape = [jax.ShapeDtypeStruct((n, c), jnp.float32),
                     jax.ShapeDtypeStruct((1, 1), jnp.float32)]
    return pl.pallas_call(
        functools.partial(_post_body, nc=nc, outer=outer is not None,
                          final=final is not None),
        grid=(n // ROW_BLK,),
        in_specs=specs,
        out_specs=out_specs,
        out_shape=out_shape,
    )(*args)


@functools.lru_cache(maxsize=None)
def _make_scatter(nc2):
    half = nc2 // 2
    mesh = plsc.VectorSubcoreMesh(core_axis_name="c", subcore_axis_name="s")

    def body(src_hbm, dst_hbm, zero_hbm, *rest):
        g_refs = rest[:nc2]
        out_refs = rest[nc2:2 * nc2]
        (sidx, didx, rows0, rows1, rows2, rows3,
         acc, g0, g1, g2, g3, isem) = rest[2 * nc2:]
        cid = lax.axis_index("c")
        sid = lax.axis_index("s")
        rbase = sid * ROWS_PER_TILE
        ebase = sid * EDGES_PER_TILE
        n_pairs = N_BLOCKS // 2

        def idx_copies(pair, buf):
            base = ebase + pair * 2 * K_EDGE
            return [
                (src_hbm.at[pl.ds(base, K_EDGE)], sidx.at[buf, 0]),
                (src_hbm.at[pl.ds(base + K_EDGE, K_EDGE)], sidx.at[buf, 1]),
                (dst_hbm.at[pl.ds(base, K_EDGE)], didx.at[buf, 0]),
                (dst_hbm.at[pl.ds(base + K_EDGE, K_EDGE)], didx.at[buf, 1]),
            ]

        def idx_prefetch(pair, buf):
            for s, d in idx_copies(pair, buf):
                pltpu.async_copy(s, d, isem)

        def idx_wait(pair, buf):
            for s, d in idx_copies(pair, buf):
                pltpu.make_async_copy(s, d, isem).wait()

        def do_chunk(g_hbm, out_hbm):
            def gather(slot, half, buf, sem):
                return pltpu.async_copy(g_hbm.at[sidx.at[slot, half]],
                                        buf, sem)

            def gwait_scatter(slot, half, buf, sem):
                pltpu.make_async_copy(g_hbm.at[sidx.at[slot, half]],
                                      buf, sem).wait()
                pltpu.sync_copy(buf, acc.at[didx.at[slot, half]], add=True)

            pltpu.sync_copy(zero_hbm, acc.at[pl.ds(rbase, ROWS_PER_TILE)])
            plsc.subcore_barrier()

            idx_prefetch(0, 0)
            idx_wait(0, 0)
            gather(0, 0, rows0, g0)
            gather(0, 1, rows1, g1)
            idx_prefetch(1, 1)

            def blk(i, carry):
                p0 = 2 * i
                idx_wait(p0 + 1, 1)
                gather(1, 0, rows2, g2)
                gather(1, 1, rows3, g3)
                gwait_scatter(0, 0, rows0, g0)
                gwait_scatter(0, 1, rows1, g1)

                @pl.when(p0 + 2 < n_pairs)
                def _():
                    idx_prefetch(p0 + 2, 0)
                    idx_wait(p0 + 2, 0)
                    gather(0, 0, rows0, g0)
                    gather(0, 1, rows1, g1)

                gwait_scatter(1, 0, rows2, g2)
                gwait_scatter(1, 1, rows3, g3)

                @pl.when(p0 + 3 < n_pairs)
                def _():
                    idx_prefetch(p0 + 3, 1)
                return carry

            lax.fori_loop(0, n_pairs // 2, blk, 0)
            gwait_scatter(0, 0, rows0, g0)
            gwait_scatter(0, 1, rows1, g1)
            plsc.subcore_barrier()
            pltpu.sync_copy(acc.at[pl.ds(rbase, ROWS_PER_TILE)],
                            out_hbm.at[pl.ds(rbase, ROWS_PER_TILE)])
            plsc.subcore_barrier()

        for side in range(2):
            @pl.when(cid == side)
            def _(side=side):
                for j in range(half):
                    k = side * half + j
                    do_chunk(g_refs[k], out_refs[k])

    return pl.kernel(
        body,
        out_type=[jax.ShapeDtypeStruct((N_PAD, FC), jnp.float32)] * nc2,
        mesh=mesh,
        scratch_types=[
            pltpu.VMEM((2, 2, K_EDGE), jnp.int32),
            pltpu.VMEM((2, 2, K_EDGE), jnp.int32),
            pltpu.VMEM((K_EDGE, FC), jnp.float32),
            pltpu.VMEM((K_EDGE, FC), jnp.float32),
            pltpu.VMEM((K_EDGE, FC), jnp.float32),
            pltpu.VMEM((K_EDGE, FC), jnp.float32),
            pltpu.VMEM_SHARED((N_PAD, FC), jnp.float32),
            pltpu.SemaphoreType.DMA,
            pltpu.SemaphoreType.DMA,
            pltpu.SemaphoreType.DMA,
            pltpu.SemaphoreType.DMA,
            pltpu.SemaphoreType.DMA,
        ],
    )


def _genconv(x, m, src, dst, zeros, p, outer, final=None):
    c = x.shape[1]
    nc = c // 128
    t = p['t'].reshape(1, 1)
    g_list = _pre(x, t, m)
    outs = _make_scatter(2 * nc)(src, dst, zeros, *g_list)
    return _post(outs[:nc], outs[nc:], x, p, outer, final)


def kernel(x, edge_index, params):
    src = edge_index[0].astype(jnp.int32)
    dst = edge_index[1].astype(jnp.int32)
    zeros = jnp.zeros((ROWS_PER_TILE, FC), jnp.float32)
    ncls = params['Wlin'].shape[1]
    wpad = jnp.zeros((1024, 128), jnp.float32).at[:, :ncls].set(params['Wlin'])
    bpad = jnp.zeros((1, 128), jnp.float32).at[0, :ncls].set(params['blin'])

    cur, m = _linear(x, params['W0'], params['b0'])
    cur, m = _genconv(cur, m, src, dst, zeros, params['conv1'], None)
    for i in range(2):
        p = params['conv%d' % (i + 1)]
        outer = (params['norm%d_g' % (i + 1)], params['norm%d_b' % (i + 1)])
        h, mh = _genconv(cur, m, src, dst, zeros, p, outer)
        cur = jnp.concatenate([cur, h], axis=-1)
        m = jnp.maximum(m, mh)
    xx, lg = _genconv(cur, m, src, dst, zeros, params['conv3'],
                      (params['norm3_g'], params['norm3_b']),
                      (params['Ng'], params['Nb'], wpad, bpad))
    return (lg[:, :ncls], xx)


# speedup vs baseline: 15.3347x; 1.0764x over previous
import functools

import jax
import jax.numpy as jnp
from jax import lax
from jax.experimental import pallas as pl
from jax.experimental.pallas import tpu as pltpu
from jax.experimental.pallas import tpu_sc as plsc

N_NODES = 10000
N_EDGES = 320000
FC = 128
ROW_BLK = 1000

N_TILES = 16
EDGES_PER_TILE = N_EDGES // N_TILES
K_EDGE = 80
N_BLOCKS = EDGES_PER_TILE // K_EDGE
N_PAD = 10240
ROWS_PER_TILE = N_PAD // N_TILES


def _dot(a, b):
    return jax.lax.dot_general(
        a, b, (((1,), (0,)), ((), ())),
        precision=jax.lax.Precision.DEFAULT,
        preferred_element_type=jnp.float32)


def _ln(h, g, b):
    mu = jnp.mean(h, axis=-1, keepdims=True)
    var = jnp.mean((h - mu) ** 2, axis=-1, keepdims=True)
    return (h - mu) / jnp.sqrt(var + 1e-5) * g + b


def _accum_max(o_ref, y):
    i = pl.program_id(0)
    m = jnp.max(jnp.maximum(y, 0.0)).reshape(1, 1)

    @pl.when(i == 0)
    def _():
        o_ref[...] = m

    @pl.when(i > 0)
    def _():
        o_ref[...] = jnp.maximum(o_ref[...], m)


def _linear_body(x_ref, w_ref, b_ref, o_ref, m_ref):
    y = _dot(x_ref[...], w_ref[...]) + b_ref[...]
    o_ref[...] = y
    _accum_max(m_ref, y)


def _linear(x, w, b):
    n, cin = x.shape
    cout = w.shape[1]
    grid = n // ROW_BLK
    return pl.pallas_call(
        _linear_body,
        grid=(grid,),
        in_specs=[
            pl.BlockSpec((ROW_BLK, cin), lambda i: (i, 0)),
            pl.BlockSpec((cin, cout), lambda i: (0, 0)),
            pl.BlockSpec((1, cout), lambda i: (0, 0)),
        ],
        out_specs=[pl.BlockSpec((ROW_BLK, cout), lambda i: (i, 0)),
                   pl.BlockSpec((1, 1), lambda i: (0, 0))],
        out_shape=[jax.ShapeDtypeStruct((n, cout), jnp.float32),
                   jax.ShapeDtypeStruct((1, 1), jnp.float32)],
    )(x, w, b.reshape(1, cout))


def _pre_body(x_ref, t_ref, m_ref, *o_refs):
    nc = len(o_refs) // 2
    r = jnp.maximum(x_ref[...], 0.0)
    msg = r + 1e-7
    e = jnp.exp((r - m_ref[...]) * t_ref[...])
    p = msg * e
    for k in range(nc):
        o_refs[k][...] = e[:, k * 128:(k + 1) * 128]
        o_refs[nc + k][...] = p[:, k * 128:(k + 1) * 128]


def _pre(x, t, m):
    n, c = x.shape
    nc = c // 128
    outs = pl.pallas_call(
        _pre_body,
        grid=(n // ROW_BLK,),
        in_specs=[
            pl.BlockSpec((ROW_BLK, c), lambda i: (i, 0)),
            pl.BlockSpec((1, 1), lambda i: (0, 0)),
            pl.BlockSpec((1, 1), lambda i: (0, 0)),
        ],
        out_specs=[pl.BlockSpec((ROW_BLK, 128), lambda i: (i, 0))] * (2 * nc),
        out_shape=[jax.ShapeDtypeStruct((n, 128), jnp.float32)] * (2 * nc),
    )(x, t, m)
    return outs


def _post_body(*refs, nc, outer, final):
    den_refs = refs[:nc]
    num_refs = refs[nc:2 * nc]
    idx = 2 * nc
    x_ref, w1_ref, b1_ref, lng_ref, lnb_ref, w2_ref, b2_ref = refs[idx:idx + 7]
    idx += 7
    if outer:
        og_ref, ob_ref = refs[idx:idx + 2]
        idx += 2
    if final:
        ng_ref, nb_ref, wl_ref, bl_ref = refs[idx:idx + 4]
        idx += 4
        xx_ref, lg_ref = refs[idx], refs[idx + 1]
    else:
        o_ref, m_ref = refs[idx], refs[idx + 1]

    den = jnp.concatenate([r[...] for r in den_refs], axis=-1)
    num = jnp.concatenate([r[...] for r in num_refs], axis=-1)
    agg = num / (den + 1e-16)
    out = agg + x_ref[...]
    h = _dot(out, w1_ref[...]) + b1_ref[...]
    h = _ln(h, lng_ref[...], lnb_ref[...])
    h = jnp.maximum(h, 0.0)
    y = _dot(h, w2_ref[...]) + b2_ref[...]
    if outer:
        y = jnp.maximum(_ln(y, og_ref[...], ob_ref[...]), 0.0)
    if final:
        cat = jnp.concatenate([x_ref[...], y], axis=-1)
        z = jnp.maximum(_ln(cat, ng_ref[...], nb_ref[...]), 0.0)
        xx_ref[...] = z
        lg_ref[...] = _dot(z, wl_ref[...]) + bl_ref[...]
    else:
        o_ref[...] = y
        _accum_max(m_ref, y)


def _post(den_chunks, num_chunks, x, p, outer, final=None):
    n, c = x.shape
    nc = c // 128
    c2 = 2 * c
    args = list(den_chunks) + list(num_chunks) + [
        x, p['W1'], p['b1'].reshape(1, c2), p['lng'].reshape(1, c2),
        p['lnb'].reshape(1, c2), p['W2'], p['b2'].reshape(1, c)]
    specs = (
        [pl.BlockSpec((ROW_BLK, 128), lambda i: (i, 0))] * (2 * nc) + [
            pl.BlockSpec((ROW_BLK, c), lambda i: (i, 0)),
            pl.BlockSpec((c, c2), lambda i: (0, 0)),
            pl.BlockSpec((1, c2), lambda i: (0, 0)),
            pl.BlockSpec((1, c2), lambda i: (0, 0)),
            pl.BlockSpec((1, c2), lambda i: (0, 0)),
            pl.BlockSpec((c2, c), lambda i: (0, 0)),
            pl.BlockSpec((1, c), lambda i: (0, 0)),
        ])
    if outer is not None:
        og, ob = outer
        args += [og.reshape(1, c), ob.reshape(1, c)]
        specs += [pl.BlockSpec((1, c), lambda i: (0, 0)),
                  pl.BlockSpec((1, c), lambda i: (0, 0))]
    if final is not None:
        ng, nb, wpad, bpad = final
        cc = 2 * c
        args += [ng.reshape(1, cc), nb.reshape(1, cc), wpad, bpad]
        specs += [pl.BlockSpec((1, cc), lambda i: (0, 0)),
                  pl.BlockSpec((1, cc), lambda i: (0, 0)),
                  pl.BlockSpec((cc, 128), lambda i: (0, 0)),
                  pl.BlockSpec((1, 128), lambda i: (0, 0))]
        out_specs = [pl.BlockSpec((ROW_BLK, cc), lambda i: (i, 0)),
                     pl.BlockSpec((ROW_BLK, 128), lambda i: (i, 0))]
        out_shape = [jax.ShapeDtypeStruct((n, cc), jnp.float32),
                     jax.ShapeDtypeStruct((n, 128), jnp.float32)]
    else:
        out_specs = [pl.BlockSpec((ROW_BLK, c), lambda i: (i, 0)),
                     pl.BlockSpec((1, 1), lambda i: (0, 0))]
        out_shape = [jax.ShapeDtypeStruct((n, c), jnp.float32),
                     jax.ShapeDtypeStruct((1, 1), jnp.float32)]
    return pl.pallas_call(
        functools.partial(_post_body, nc=nc, outer=outer is not None,
                          final=final is not None),
        grid=(n // ROW_BLK,),
        in_specs=specs,
        out_specs=out_specs,
        out_shape=out_shape,
    )(*args)


@functools.lru_cache(maxsize=None)
def _make_scatter(nc2):
    half = nc2 // 2
    mesh = plsc.VectorSubcoreMesh(core_axis_name="c", subcore_axis_name="s")

    def body(src_hbm, dst_hbm, zero_hbm, *rest):
        g_refs = rest[:nc2]
        out_refs = rest[nc2:2 * nc2]
        (sidx, didx, rows0, rows1, rows2, rows3,
         acc, g0, g1, g2, g3, isem, ssem) = rest[2 * nc2:]
        cid = lax.axis_index("c")
        sid = lax.axis_index("s")
        rbase = sid * ROWS_PER_TILE
        ebase = sid * EDGES_PER_TILE
        n_pairs = N_BLOCKS // 2

        def idx_copies(pair, buf):
            base = ebase + pair * 2 * K_EDGE
            return [
                (src_hbm.at[pl.ds(base, K_EDGE)], sidx.at[buf, 0]),
                (src_hbm.at[pl.ds(base + K_EDGE, K_EDGE)], sidx.at[buf, 1]),
                (dst_hbm.at[pl.ds(base, K_EDGE)], didx.at[buf, 0]),
                (dst_hbm.at[pl.ds(base + K_EDGE, K_EDGE)], didx.at[buf, 1]),
            ]

        def idx_prefetch(pair, buf):
            for s, d in idx_copies(pair, buf):
                pltpu.async_copy(s, d, isem)

        def idx_wait(pair, buf):
            for s, d in idx_copies(pair, buf):
                pltpu.make_async_copy(s, d, isem).wait()

        def do_chunk(g_hbm, out_hbm):
            def gather(slot, half, buf, sem):
                return pltpu.async_copy(g_hbm.at[sidx.at[slot, half]],
                                        buf, sem)

            def gwait_scatter(slot, half, buf, sem):
                pltpu.make_async_copy(g_hbm.at[sidx.at[slot, half]],
                                      buf, sem).wait()
                pltpu.sync_copy(buf, acc.at[didx.at[slot, half]], add=True)

            pltpu.sync_copy(zero_hbm, acc.at[pl.ds(rbase, ROWS_PER_TILE)])
            plsc.subcore_barrier()

            idx_prefetch(0, 0)
            idx_wait(0, 0)
            gather(0, 0, rows0, g0)
            gather(0, 1, rows1, g1)
            idx_prefetch(1, 1)

            def gwait(slot, half, buf, sem):
                pltpu.make_async_copy(g_hbm.at[sidx.at[slot, half]],
                                      buf, sem).wait()

            def scat_async(slot, half, buf):
                pltpu.async_copy(buf, acc.at[didx.at[slot, half]], ssem,
                                 add=True)

            def scat_drain(slot, half, buf):
                pltpu.make_async_copy(buf, acc.at[didx.at[slot, half]],
                                      ssem).wait()

            def blk(i, carry):
                p0 = 2 * i
                idx_wait(p0 + 1, 1)
                gather(1, 0, rows2, g2)
                gather(1, 1, rows3, g3)
                gwait(0, 0, rows0, g0)
                scat_async(0, 0, rows0)
                gwait(0, 1, rows1, g1)
                scat_async(0, 1, rows1)

                @pl.when(p0 + 2 < n_pairs)
                def _():
                    idx_prefetch(p0 + 2, 0)
                    idx_wait(p0 + 2, 0)
                    scat_drain(0, 0, rows0)
                    scat_drain(0, 1, rows1)
                    gather(0, 0, rows0, g0)
                    gather(0, 1, rows1, g1)

                gwait(1, 0, rows2, g2)
                scat_async(1, 0, rows2)
                gwait(1, 1, rows3, g3)
                scat_async(1, 1, rows3)

                @pl.when(p0 + 3 < n_pairs)
                def _():
                    idx_prefetch(p0 + 3, 1)
                scat_drain(1, 0, rows2)
                scat_drain(1, 1, rows3)
                return carry

            lax.fori_loop(0, n_pairs // 2, blk, 0)
            gwait_scatter(0, 0, rows0, g0)
            gwait_scatter(0, 1, rows1, g1)
            plsc.subcore_barrier()
            pltpu.sync_copy(acc.at[pl.ds(rbase, ROWS_PER_TILE)],
                            out_hbm.at[pl.ds(rbase, ROWS_PER_TILE)])
            plsc.subcore_barrier()

        for side in range(2):
            @pl.when(cid == side)
            def _(side=side):
                for j in range(half):
                    k = side * half + j
                    do_chunk(g_refs[k], out_refs[k])

    return pl.kernel(
        body,
        out_type=[jax.ShapeDtypeStruct((N_PAD, FC), jnp.float32)] * nc2,
        mesh=mesh,
        scratch_types=[
            pltpu.VMEM((2, 2, K_EDGE), jnp.int32),
            pltpu.VMEM((2, 2, K_EDGE), jnp.int32),
            pltpu.VMEM((K_EDGE, FC), jnp.float32),
            pltpu.VMEM((K_EDGE, FC), jnp.float32),
            pltpu.VMEM((K_EDGE, FC), jnp.float32),
            pltpu.VMEM((K_EDGE, FC), jnp.float32),
            pltpu.VMEM_SHARED((N_PAD, FC), jnp.float32),
            pltpu.SemaphoreType.DMA,
            pltpu.SemaphoreType.DMA,
            pltpu.SemaphoreType.DMA,
            pltpu.SemaphoreType.DMA,
            pltpu.SemaphoreType.DMA,
            pltpu.SemaphoreType.DMA,
        ],
    )


def _genconv(x, m, src, dst, zeros, p, outer, final=None):
    c = x.shape[1]
    nc = c // 128
    t = p['t'].reshape(1, 1)
    g_list = _pre(x, t, m)
    outs = _make_scatter(2 * nc)(src, dst, zeros, *g_list)
    return _post(outs[:nc], outs[nc:], x, p, outer, final)


def kernel(x, edge_index, params):
    src = edge_index[0].astype(jnp.int32)
    dst = edge_index[1].astype(jnp.int32)
    zeros = jnp.zeros((ROWS_PER_TILE, FC), jnp.float32)
    ncls = params['Wlin'].shape[1]
    wpad = jnp.zeros((1024, 128), jnp.float32).at[:, :ncls].set(params['Wlin'])
    bpad = jnp.zeros((1, 128), jnp.float32).at[0, :ncls].set(params['blin'])

    cur, m = _linear(x, params['W0'], params['b0'])
    cur, m = _genconv(cur, m, src, dst, zeros, params['conv1'], None)
    for i in range(2):
        p = params['conv%d' % (i + 1)]
        outer = (params['norm%d_g' % (i + 1)], params['norm%d_b' % (i + 1)])
        h, mh = _genconv(cur, m, src, dst, zeros, p, outer)
        cur = jnp.concatenate([cur, h], axis=-1)
        m = jnp.maximum(m, mh)
    xx, lg = _genconv(cur, m, src, dst, zeros, params['conv3'],
                      (params['norm3_g'], params['norm3_b']),
                      (params['Ng'], params['Nb'], wpad, bpad))
    return (lg[:, :ncls], xx)
